# Initial kernel scaffold; baseline (speedup 1.0000x reference)
import functools

import jax
import jax.numpy as jnp
from jax import lax
from jax.experimental import pallas as pl
from jax.experimental.pallas import tpu as pltpu
from jax.experimental.pallas import tpu_sc as plsc

N = 10000
E = 320000
D = 128
W = 144
NC, NS = 2, 16
NW = NC * NS
EW = E // NW
CH = 80
NCHUNK = EW // CH
NPAD = 10240
RPT = NPAD // NS

BN = 400
BE = 512

_MESH = plsc.VectorSubcoreMesh(
    core_axis_name="c", subcore_axis_name="s", num_cores=NC, num_subcores=NS)


def _prep_body(h_ref, pp_ref, wa_ref, wb_ref, b1_ref, t1_ref, t2_ref):
    h = h_ref[...]
    a = jnp.dot(h, wa_ref[...], preferred_element_type=jnp.float32) + b1_ref[...]
    b = jnp.dot(h, wb_ref[...], preferred_element_type=jnp.float32)
    pp = pp_ref[...]
    t1_ref[...] = jnp.concatenate([a, pp], axis=1)
    t2_ref[...] = jnp.concatenate([b, -pp], axis=1)


def _prep(h, pos16, wa, wb, b1):
    return pl.pallas_call(
        _prep_body,
        grid=(N // BN,),
        in_specs=[
            pl.BlockSpec((BN, D), lambda i: (i, 0)),
            pl.BlockSpec((BN, 16), lambda i: (i, 0)),
            pl.BlockSpec((D, D), lambda i: (0, 0)),
            pl.BlockSpec((D, D), lambda i: (0, 0)),
            pl.BlockSpec((1, D), lambda i: (0, 0)),
        ],
        out_specs=[
            pl.BlockSpec((BN, W), lambda i: (i, 0)),
            pl.BlockSpec((BN, W), lambda i: (i, 0)),
        ],
        out_shape=[jax.ShapeDtypeStruct((N, W), jnp.float32)] * 2,
    )(h, pos16, wa, wb, b1)


@functools.partial(
    pl.kernel,
    out_type=[jax.ShapeDtypeStruct((E, W), jnp.float32),
              jax.ShapeDtypeStruct((E, W), jnp.float32)],
    mesh=_MESH,
    scratch_types=[
        pltpu.VMEM((CH,), jnp.int32),
        pltpu.VMEM((CH,), jnp.int32),
        pltpu.VMEM((CH, W), jnp.float32),
        pltpu.VMEM((CH, W), jnp.float32),
        pltpu.SemaphoreType.DMA,
        pltpu.SemaphoreType.DMA,
    ],
)
def _sc_gather(t1_hbm, t2_hbm, row_hbm, col_hbm, g1_hbm, g2_hbm,
               ir_v, ic_v, r1_v, r2_v, sem1, sem2):
    wid = lax.axis_index("s") * NC + lax.axis_index("c")
    base0 = wid * EW

    def body(k, carry):
        base = base0 + k * CH
        pltpu.sync_copy(row_hbm.at[pl.ds(base, CH)], ir_v)
        pltpu.sync_copy(col_hbm.at[pl.ds(base, CH)], ic_v)
        cp1 = pltpu.async_copy(t1_hbm.at[ir_v], r1_v, sem1)
        cp2 = pltpu.async_copy(t2_hbm.at[ic_v], r2_v, sem2)
        cp1.wait()
        cp2.wait()
        pltpu.sync_copy(r1_v, g1_hbm.at[pl.ds(base, CH)])
        pltpu.sync_copy(r2_v, g2_hbm.at[pl.ds(base, CH)])
        return carry

    lax.fori_loop(0, NCHUNK, body, 0)


def _edge_body(g1_ref, g2_ref, w256_ref, we2_ref, b2_ref, wc1_ref, bc1_ref,
               wc2_ref, cnt_ref, y_ref):
    s = g1_ref[...] + g2_ref[...]
    f = s[:, :D]
    dgeo = s[:, D:]
    sq = jnp.sum(dgeo * dgeo, axis=1, keepdims=True)
    x1 = jax.nn.silu(f + sq * w256_ref[...])
    m = jax.nn.silu(jnp.dot(x1, we2_ref[...], preferred_element_type=jnp.float32)
                    + b2_ref[...])
    t = jax.nn.silu(jnp.dot(m, wc1_ref[...], preferred_element_type=jnp.float32)
                    + bc1_ref[...])
    cw = jnp.dot(t, wc2_ref[...], preferred_element_type=jnp.float32)
    scale = cw * lax.rsqrt(sq + 1e-8)
    pu = dgeo * scale + cnt_ref[...]
    y_ref[...] = jnp.concatenate([m, pu], axis=1)


def _edge(g1, g2, w256, we2, b2, wc1, bc1, wc2, cnt_row):
    return pl.pallas_call(
        _edge_body,
        grid=(E // BE,),
        in_specs=[
            pl.BlockSpec((BE, W), lambda i: (i, 0)),
            pl.BlockSpec((BE, W), lambda i: (i, 0)),
            pl.BlockSpec((1, D), lambda i: (0, 0)),
            pl.BlockSpec((D, D), lambda i: (0, 0)),
            pl.BlockSpec((1, D), lambda i: (0, 0)),
            pl.BlockSpec((D, D), lambda i: (0, 0)),
            pl.BlockSpec((1, D), lambda i: (0, 0)),
            pl.BlockSpec((D, 1), lambda i: (0, 0)),
            pl.BlockSpec((1, 16), lambda i: (0, 0)),
        ],
        out_specs=pl.BlockSpec((BE, W), lambda i: (i, 0)),
        out_shape=jax.ShapeDtypeStruct((E, W), jnp.float32),
    )(g1, g2, w256, we2, b2, wc1, bc1, wc2, cnt_row)


@functools.partial(
    pl.kernel,
    out_type=jax.ShapeDtypeStruct((NC, NPAD, W), jnp.float32),
    mesh=_MESH,
    scratch_types=[
        pltpu.VMEM((CH,), jnp.int32),
        pltpu.VMEM((CH, W), jnp.float32),
        pltpu.VMEM_SHARED((NPAD, W), jnp.float32),
    ],
)
def _sc_scatter(y_hbm, row_hbm, zero_hbm, part_hbm, iv, yb, accum):
    c = lax.axis_index("c")
    s = lax.axis_index("s")
    pltpu.sync_copy(zero_hbm.at[pl.ds(s * RPT, RPT)],
                    accum.at[pl.ds(s * RPT, RPT)])
    plsc.subcore_barrier()
    base0 = (c * NS + s) * EW

    def body(k, carry):
        base = base0 + k * CH
        pltpu.sync_copy(row_hbm.at[pl.ds(base, CH)], iv)
        pltpu.sync_copy(y_hbm.at[pl.ds(base, CH)], yb)
        pltpu.sync_copy(yb, accum.at[iv], add=True)
        return carry

    lax.fori_loop(0, NCHUNK, body, 0)
    plsc.subcore_barrier()
    pltpu.sync_copy(accum.at[pl.ds(s * RPT, RPT)],
                    part_hbm.at[c, pl.ds(s * RPT, RPT)])


def _node_body(h_ref, pos_ref, p0_ref, p1_ref, wn1a_ref, wn1b_ref, bn1_ref,
               wn2_ref, bn2_ref, ho_ref, po_ref):
    h = h_ref[...]
    p = p0_ref[...] + p1_ref[...]
    m_i = p[:, :D]
    num = p[:, D:D + 3]
    cnt = p[:, D + 3:D + 4]
    x = (jnp.dot(h, wn1a_ref[...], preferred_element_type=jnp.float32)
         + jnp.dot(m_i, wn1b_ref[...], preferred_element_type=jnp.float32)
         + bn1_ref[...])
    hu = (jnp.dot(jax.nn.silu(x), wn2_ref[...], preferred_element_type=jnp.float32)
          + bn2_ref[...])
    ho_ref[...] = h + hu
    po_ref[...] = pos_ref[...] + num / jnp.maximum(cnt, 1.0)


def _node(h, pos, p0, p1, wn1a, wn1b, bn1, wn2, bn2):
    return pl.pallas_call(
        _node_body,
        grid=(N // BN,),
        in_specs=[
            pl.BlockSpec((BN, D), lambda i: (i, 0)),
            pl.BlockSpec((BN, 3), lambda i: (i, 0)),
            pl.BlockSpec((BN, W), lambda i: (i, 0)),
            pl.BlockSpec((BN, W), lambda i: (i, 0)),
            pl.BlockSpec((D, D), lambda i: (0, 0)),
            pl.BlockSpec((D, D), lambda i: (0, 0)),
            pl.BlockSpec((1, D), lambda i: (0, 0)),
            pl.BlockSpec((D, D), lambda i: (0, 0)),
            pl.BlockSpec((1, D), lambda i: (0, 0)),
        ],
        out_specs=[
            pl.BlockSpec((BN, D), lambda i: (i, 0)),
            pl.BlockSpec((BN, 3), lambda i: (i, 0)),
        ],
        out_shape=[jax.ShapeDtypeStruct((N, D), jnp.float32),
                   jax.ShapeDtypeStruct((N, 3), jnp.float32)],
    )(h, pos, p0, p1, wn1a, wn1b, bn1, wn2, bn2)


def kernel(h, pos, edge_index, W_e1, b_e1, W_e2, b_e2, W_c1, b_c1, W_c2,
           W_n1, b_n1, W_n2, b_n2):
    row = edge_index[0].astype(jnp.int32)
    col = edge_index[1].astype(jnp.int32)
    pos16 = jnp.pad(pos, ((0, 0), (0, 13)))
    cnt_row = jnp.zeros((1, 16), jnp.float32).at[0, 3].set(1.0)

    t1, t2 = _prep(h, pos16, W_e1[:D], W_e1[D:2 * D], b_e1.reshape(1, D))
    g1, g2 = _sc_gather(t1, t2, row, col)
    y = _edge(g1, g2, W_e1[2 * D:2 * D + 1], W_e2, b_e2.reshape(1, D),
              W_c1, b_c1.reshape(1, D), W_c2, cnt_row)
    parts = _sc_scatter(y, row, jnp.zeros((NPAD, W), jnp.float32))
    h_out, pos_out = _node(h, pos, parts[0], parts[1],
                           W_n1[:D], W_n1[D:], b_n1.reshape(1, D),
                           W_n2, b_n2.reshape(1, D))
    return h_out, pos_out


# speedup vs baseline: 3.0814x; 3.0814x over previous
import functools

import jax
import jax.numpy as jnp
from jax import lax
from jax.experimental import pallas as pl
from jax.experimental.pallas import tpu as pltpu
from jax.experimental.pallas import tpu_sc as plsc

N = 10000
E = 320000
D = 128
L = 16
NC, NS = 2, 16
NW = NC * NS
EW = E // NW
CH = 80
NG = CH // L
NCHUNK = EW // CH
NPAD = 10240
RPT = NPAD // NS
P4 = NPAD * 4
PPT = P4 // NS

BN = 400
BE = 512
EB = E // BE


@functools.cache
def _mesh():
    return plsc.VectorSubcoreMesh(
        core_axis_name="c", subcore_axis_name="s",
        num_cores=NC, num_subcores=NS)


def _prep_body(h_ref, wa_ref, wb_ref, b1_ref, t1_ref, t2_ref):
    h = h_ref[...]
    t1_ref[...] = jnp.dot(h, wa_ref[...],
                          preferred_element_type=jnp.float32) + b1_ref[...]
    t2_ref[...] = jnp.dot(h, wb_ref[...], preferred_element_type=jnp.float32)


def _prep(h, wa, wb, b1):
    return pl.pallas_call(
        _prep_body,
        grid=(N // BN,),
        in_specs=[
            pl.BlockSpec((BN, D), lambda i: (i, 0)),
            pl.BlockSpec((D, D), lambda i: (0, 0)),
            pl.BlockSpec((D, D), lambda i: (0, 0)),
            pl.BlockSpec((1, D), lambda i: (0, 0)),
        ],
        out_specs=[
            pl.BlockSpec((BN, D), lambda i: (i, 0)),
            pl.BlockSpec((BN, D), lambda i: (i, 0)),
        ],
        out_shape=[jax.ShapeDtypeStruct((N, D), jnp.float32)] * 2,
    )(h, wa, wb, b1)


@functools.cache
def _gather_kernel():
    @functools.partial(
        pl.kernel,
        out_type=[jax.ShapeDtypeStruct((E, D), jnp.float32),
                  jax.ShapeDtypeStruct((E, D), jnp.float32),
                  jax.ShapeDtypeStruct((E,), jnp.float32),
                  jax.ShapeDtypeStruct((E,), jnp.float32),
                  jax.ShapeDtypeStruct((E,), jnp.float32),
                  jax.ShapeDtypeStruct((E,), jnp.float32)],
        mesh=_mesh(),
        compiler_params=pltpu.CompilerParams(needs_layout_passes=False),
        scratch_types=[
            pltpu.VMEM((CH,), jnp.int32),
            pltpu.VMEM((CH,), jnp.int32),
            pltpu.VMEM((CH, D), jnp.float32),
            pltpu.VMEM((CH, D), jnp.float32),
            pltpu.VMEM((N,), jnp.float32),
            pltpu.VMEM((N,), jnp.float32),
            pltpu.VMEM((N,), jnp.float32),
            pltpu.VMEM((CH,), jnp.float32),
            pltpu.VMEM((CH,), jnp.float32),
            pltpu.VMEM((CH,), jnp.float32),
            pltpu.VMEM((CH,), jnp.float32),
            pltpu.SemaphoreType.DMA,
            pltpu.SemaphoreType.DMA,
        ],
    )
    def body_fn(t1_hbm, t2_hbm, row_hbm, col_hbm, px_hbm, py_hbm, pz_hbm,
                g1_hbm, g2_hbm, dx_hbm, dy_hbm, dz_hbm, sq_hbm,
                ir_v, ic_v, r1_v, r2_v, px_v, py_v, pz_v,
                dx_v, dy_v, dz_v, sq_v, sem1, sem2):
        wid = lax.axis_index("s") * NC + lax.axis_index("c")
        base0 = wid * EW
        pltpu.sync_copy(px_hbm, px_v)
        pltpu.sync_copy(py_hbm, py_v)
        pltpu.sync_copy(pz_hbm, pz_v)

        def body(k, carry):
            base = base0 + k * CH
            pltpu.sync_copy(row_hbm.at[pl.ds(base, CH)], ir_v)
            pltpu.sync_copy(col_hbm.at[pl.ds(base, CH)], ic_v)
            cp1 = pltpu.async_copy(t1_hbm.at[ir_v], r1_v, sem1)
            cp2 = pltpu.async_copy(t2_hbm.at[ic_v], r2_v, sem2)
            for j in range(NG):
                sl = pl.ds(j * L, L)
                ivr = ir_v[sl]
                ivc = ic_v[sl]
                dx = (plsc.load_gather(px_v, [ivr])
                      - plsc.load_gather(px_v, [ivc]))
                dy = (plsc.load_gather(py_v, [ivr])
                      - plsc.load_gather(py_v, [ivc]))
                dz = (plsc.load_gather(pz_v, [ivr])
                      - plsc.load_gather(pz_v, [ivc]))
                dx_v[sl] = dx
                dy_v[sl] = dy
                dz_v[sl] = dz
                sq_v[sl] = dx * dx + dy * dy + dz * dz
            cp1.wait()
            cp2.wait()
            pltpu.sync_copy(r1_v, g1_hbm.at[pl.ds(base, CH)])
            pltpu.sync_copy(r2_v, g2_hbm.at[pl.ds(base, CH)])
            pltpu.sync_copy(dx_v, dx_hbm.at[pl.ds(base, CH)])
            pltpu.sync_copy(dy_v, dy_hbm.at[pl.ds(base, CH)])
            pltpu.sync_copy(dz_v, dz_hbm.at[pl.ds(base, CH)])
            pltpu.sync_copy(sq_v, sq_hbm.at[pl.ds(base, CH)])
            return carry

        lax.fori_loop(0, NCHUNK, body, 0)

    return body_fn


def _sc_gather(t1, t2, row, col, px, py, pz):
    return _gather_kernel()(t1, t2, row, col, px, py, pz)


def _edge_body(g1_ref, g2_ref, dx_ref, dy_ref, dz_ref, sq_ref,
               w256_ref, we2_ref, b2_ref, wc1_ref, bc1_ref, wc2_ref,
               m_ref, px_ref, py_ref, pz_ref):
    f = g1_ref[...] + g2_ref[...]
    sq = sq_ref[0].reshape(BE, 1)
    x1 = jax.nn.silu(f + sq * w256_ref[...])
    m = jax.nn.silu(jnp.dot(x1, we2_ref[...], preferred_element_type=jnp.float32)
                    + b2_ref[...])
    t = jax.nn.silu(jnp.dot(m, wc1_ref[...], preferred_element_type=jnp.float32)
                    + bc1_ref[...])
    cw = jnp.dot(t, wc2_ref[...], preferred_element_type=jnp.float32)
    scale = (cw * lax.rsqrt(sq + 1e-8)).reshape(1, 1, BE)
    m_ref[...] = m
    px_ref[...] = dx_ref[...] * scale
    py_ref[...] = dy_ref[...] * scale
    pz_ref[...] = dz_ref[...] * scale


def _edge(g1, g2, dxr, dyr, dzr, sqr, w256, we2, b2, wc1, bc1, wc2):
    row_spec = pl.BlockSpec((1, 1, BE), lambda i: (i, 0, 0))
    full = lambda shape: pl.BlockSpec(shape, lambda i: (0, 0))
    return pl.pallas_call(
        _edge_body,
        grid=(EB,),
        in_specs=[
            pl.BlockSpec((BE, D), lambda i: (i, 0)),
            pl.BlockSpec((BE, D), lambda i: (i, 0)),
            row_spec, row_spec, row_spec, row_spec,
            full((1, D)), full((D, D)), full((1, D)),
            full((D, D)), full((1, D)), full((D, 1)),
        ],
        out_specs=[
            pl.BlockSpec((BE, D), lambda i: (i, 0)),
            row_spec, row_spec, row_spec,
        ],
        out_shape=[jax.ShapeDtypeStruct((E, D), jnp.float32),
                   jax.ShapeDtypeStruct((EB, 1, BE), jnp.float32),
                   jax.ShapeDtypeStruct((EB, 1, BE), jnp.float32),
                   jax.ShapeDtypeStruct((EB, 1, BE), jnp.float32)],
    )(g1, g2, dxr, dyr, dzr, sqr, w256, we2, b2, wc1, bc1, wc2)


@functools.cache
def _scatter_kernel():
    @functools.partial(
        pl.kernel,
        out_type=jax.ShapeDtypeStruct((NC, NPAD, D), jnp.float32),
        mesh=_mesh(),
        scratch_types=[
            pltpu.VMEM((CH,), jnp.int32),
            pltpu.VMEM((CH, D), jnp.float32),
            pltpu.VMEM_SHARED((NPAD, D), jnp.float32),
        ],
    )
    def body_fn(m_hbm, row_hbm, z_hbm, pm_hbm, iv, mb, accum):
        c = lax.axis_index("c")
        s = lax.axis_index("s")
        pltpu.sync_copy(z_hbm.at[pl.ds(s * RPT, RPT)],
                        accum.at[pl.ds(s * RPT, RPT)])
        plsc.subcore_barrier()
        base0 = (c * NS + s) * EW

        def body(k, carry):
            base = base0 + k * CH
            pltpu.sync_copy(row_hbm.at[pl.ds(base, CH)], iv)
            pltpu.sync_copy(m_hbm.at[pl.ds(base, CH)], mb)
            pltpu.sync_copy(mb, accum.at[iv], add=True)
            return carry

        lax.fori_loop(0, NCHUNK, body, 0)
        plsc.subcore_barrier()
        pltpu.sync_copy(accum.at[pl.ds(s * RPT, RPT)],
                        pm_hbm.at[c, pl.ds(s * RPT, RPT)])

    return body_fn


def _sc_scatter(m, row, zeros2d):
    return _scatter_kernel()(m, row, zeros2d)


@functools.cache
def _pos_scatter_kernel():
    @functools.partial(
        pl.kernel,
        out_type=jax.ShapeDtypeStruct((NW, P4), jnp.float32),
        mesh=_mesh(),
        compiler_params=pltpu.CompilerParams(needs_layout_passes=False),
        scratch_types=[
            pltpu.VMEM((CH,), jnp.int32),
            pltpu.VMEM((CH,), jnp.float32),
            pltpu.VMEM((CH,), jnp.float32),
            pltpu.VMEM((CH,), jnp.float32),
            pltpu.VMEM((P4,), jnp.float32),
        ],
    )
    def body_fn(row_hbm, pux_hbm, puy_hbm, puz_hbm, z4_hbm, pp_hbm,
                iv, pxb, pyb, pzb, pacc):
        c = lax.axis_index("c")
        s = lax.axis_index("s")
        pltpu.sync_copy(z4_hbm, pacc)
        base0 = (c * NS + s) * EW
        ones = jnp.ones((L,), jnp.float32)

        def body(k, carry):
            base = base0 + k * CH
            pltpu.sync_copy(row_hbm.at[pl.ds(base, CH)], iv)
            pltpu.sync_copy(pux_hbm.at[pl.ds(base, CH)], pxb)
            pltpu.sync_copy(puy_hbm.at[pl.ds(base, CH)], pyb)
            pltpu.sync_copy(puz_hbm.at[pl.ds(base, CH)], pzb)
            for j in range(NG):
                sl = pl.ds(j * L, L)
                i4 = iv[sl] * 4
                plsc.addupdate_scatter(pacc, [i4], pxb[sl])
                plsc.addupdate_scatter(pacc, [i4 + 1], pyb[sl])
                plsc.addupdate_scatter(pacc, [i4 + 2], pzb[sl])
                plsc.addupdate_scatter(pacc, [i4 + 3], ones)
            return carry

        lax.fori_loop(0, NCHUNK, body, 0)
        pltpu.sync_copy(pacc, pp_hbm.at[c * NS + s])

    return body_fn


def _sc_pos_scatter(row, pux, puy, puz, zeros4):
    return _pos_scatter_kernel()(row, pux, puy, puz, zeros4)


def _node_body(h_ref, pos_ref, pm0_ref, pm1_ref, pp_ref,
               wn1a_ref, wn1b_ref, bn1_ref, wn2_ref, bn2_ref, ho_ref, po_ref):
    h = h_ref[...]
    m_i = pm0_ref[...] + pm1_ref[...]
    q = jnp.sum(pp_ref[...], axis=0)
    num = q[:, :3]
    cnt = q[:, 3:4]
    x = (jnp.dot(h, wn1a_ref[...], preferred_element_type=jnp.float32)
         + jnp.dot(m_i, wn1b_ref[...], preferred_element_type=jnp.float32)
         + bn1_ref[...])
    hu = (jnp.dot(jax.nn.silu(x), wn2_ref[...], preferred_element_type=jnp.float32)
          + bn2_ref[...])
    ho_ref[...] = h + hu
    po_ref[...] = pos_ref[...] + num / jnp.maximum(cnt, 1.0)


def _node(h, pos, pm0, pm1, pp, wn1a, wn1b, bn1, wn2, bn2):
    return pl.pallas_call(
        _node_body,
        grid=(N // BN,),
        in_specs=[
            pl.BlockSpec((BN, D), lambda i: (i, 0)),
            pl.BlockSpec((BN, 3), lambda i: (i, 0)),
            pl.BlockSpec((BN, D), lambda i: (i, 0)),
            pl.BlockSpec((BN, D), lambda i: (i, 0)),
            pl.BlockSpec((NW, BN, 4), lambda i: (0, i, 0)),
            pl.BlockSpec((D, D), lambda i: (0, 0)),
            pl.BlockSpec((D, D), lambda i: (0, 0)),
            pl.BlockSpec((1, D), lambda i: (0, 0)),
            pl.BlockSpec((D, D), lambda i: (0, 0)),
            pl.BlockSpec((1, D), lambda i: (0, 0)),
        ],
        out_specs=[
            pl.BlockSpec((BN, D), lambda i: (i, 0)),
            pl.BlockSpec((BN, 3), lambda i: (i, 0)),
        ],
        out_shape=[jax.ShapeDtypeStruct((N, D), jnp.float32),
                   jax.ShapeDtypeStruct((N, 3), jnp.float32)],
    )(h, pos, pm0, pm1, pp, wn1a, wn1b, bn1, wn2, bn2)


def kernel(h, pos, edge_index, W_e1, b_e1, W_e2, b_e2, W_c1, b_c1, W_c2,
           W_n1, b_n1, W_n2, b_n2):
    row = edge_index[0].astype(jnp.int32)
    col = edge_index[1].astype(jnp.int32)
    px = pos[:, 0]
    py = pos[:, 1]
    pz = pos[:, 2]

    t1, t2 = _prep(h, W_e1[:D], W_e1[D:2 * D], b_e1.reshape(1, D))
    g1, g2, dxa, dya, dza, sqa = _sc_gather(t1, t2, row, col, px, py, pz)
    m, pux, puy, puz = _edge(
        g1, g2, dxa.reshape(EB, 1, BE), dya.reshape(EB, 1, BE),
        dza.reshape(EB, 1, BE), sqa.reshape(EB, 1, BE),
        W_e1[2 * D:2 * D + 1], W_e2, b_e2.reshape(1, D),
        W_c1, b_c1.reshape(1, D), W_c2)
    pm = _sc_scatter(m, row, jnp.zeros((NPAD, D), jnp.float32))
    pp = _sc_pos_scatter(row, pux.reshape(E), puy.reshape(E),
                         puz.reshape(E), jnp.zeros((P4,), jnp.float32))
    pp = pp.reshape(NW, NPAD, 4)
    h_out, pos_out = _node(h, pos, pm[0], pm[1], pp,
                           W_n1[:D], W_n1[D:], b_n1.reshape(1, D),
                           W_n2, b_n2.reshape(1, D))
    return h_out, pos_out


# speedup vs baseline: 4.5294x; 1.4699x over previous
import functools

import jax
import jax.numpy as jnp
from jax import lax
from jax.experimental import pallas as pl
from jax.experimental.pallas import tpu as pltpu
from jax.experimental.pallas import tpu_sc as plsc

N = 10000
E = 320000
D = 128
L = 16
NC, NS = 2, 16
NW = NC * NS
EW = E // NW
CH = 80
NG = CH // L
NCHUNK = EW // CH
CHP = 2000
NCHP = EW // CHP
NPAD = 10240
RPT = NPAD // NS
P4 = NPAD * 4
PPT = P4 // NS

BN = 400
BE = 512
EB = E // BE


@functools.cache
def _mesh():
    return plsc.VectorSubcoreMesh(
        core_axis_name="c", subcore_axis_name="s",
        num_cores=NC, num_subcores=NS)


def _prep_body(h_ref, wa_ref, wb_ref, b1_ref, t1_ref, t2_ref):
    h = h_ref[...]
    t1_ref[...] = jnp.dot(h, wa_ref[...],
                          preferred_element_type=jnp.float32) + b1_ref[...]
    t2_ref[...] = jnp.dot(h, wb_ref[...], preferred_element_type=jnp.float32)


def _prep(h, wa, wb, b1):
    return pl.pallas_call(
        _prep_body,
        grid=(N // BN,),
        in_specs=[
            pl.BlockSpec((BN, D), lambda i: (i, 0)),
            pl.BlockSpec((D, D), lambda i: (0, 0)),
            pl.BlockSpec((D, D), lambda i: (0, 0)),
            pl.BlockSpec((1, D), lambda i: (0, 0)),
        ],
        out_specs=[
            pl.BlockSpec((BN, D), lambda i: (i, 0)),
            pl.BlockSpec((BN, D), lambda i: (i, 0)),
        ],
        out_shape=[jax.ShapeDtypeStruct((N, D), jnp.float32)] * 2,
    )(h, wa, wb, b1)


NB = 3


@functools.cache
def _gather_kernel():
    @functools.partial(
        pl.kernel,
        out_type=[jax.ShapeDtypeStruct((E, D), jnp.float32),
                  jax.ShapeDtypeStruct((E, D), jnp.float32),
                  jax.ShapeDtypeStruct((E,), jnp.float32),
                  jax.ShapeDtypeStruct((E,), jnp.float32),
                  jax.ShapeDtypeStruct((E,), jnp.float32),
                  jax.ShapeDtypeStruct((E,), jnp.float32)],
        mesh=_mesh(),
        compiler_params=pltpu.CompilerParams(needs_layout_passes=False),
        scratch_types=[
            pltpu.VMEM((EW,), jnp.int32),
            pltpu.VMEM((EW,), jnp.int32),
            pltpu.VMEM((N,), jnp.float32),
            pltpu.VMEM((N,), jnp.float32),
            pltpu.VMEM((N,), jnp.float32),
            [pltpu.VMEM((CH, D), jnp.float32)] * NB,
            [pltpu.VMEM((CH, D), jnp.float32)] * NB,
            [pltpu.VMEM((4, CH), jnp.float32)] * NB,
            [pltpu.SemaphoreType.DMA] * NB,
            [pltpu.SemaphoreType.DMA] * NB,
        ],
    )
    def body_fn(t1_hbm, t2_hbm, row_hbm, col_hbm, px_hbm, py_hbm, pz_hbm,
                g1_hbm, g2_hbm, dx_hbm, dy_hbm, dz_hbm, sq_hbm,
                ir_v, ic_v, px_v, py_v, pz_v, r1s, r2s, gxs, sgs, sos):
        geo_hbms = (dx_hbm, dy_hbm, dz_hbm, sq_hbm)
        wid = lax.axis_index("s") * NC + lax.axis_index("c")
        base0 = wid * EW
        pltpu.sync_copy(row_hbm.at[pl.ds(base0, EW)], ir_v)
        pltpu.sync_copy(col_hbm.at[pl.ds(base0, EW)], ic_v)
        pltpu.sync_copy(px_hbm, px_v)
        pltpu.sync_copy(py_hbm, py_v)
        pltpu.sync_copy(pz_hbm, pz_v)

        def start(k, b):
            off = k * CH
            pltpu.async_copy(t1_hbm.at[ir_v.at[pl.ds(off, CH)]], r1s[b], sgs[b])
            pltpu.async_copy(t2_hbm.at[ic_v.at[pl.ds(off, CH)]], r2s[b], sgs[b])

        def geom(k, b):
            gx = gxs[b]
            for j in range(NG):
                sl = pl.ds(k * CH + j * L, L)
                osl = pl.ds(j * L, L)
                ivr = ir_v[sl]
                ivc = ic_v[sl]
                dx = (plsc.load_gather(px_v, [ivr])
                      - plsc.load_gather(px_v, [ivc]))
                dy = (plsc.load_gather(py_v, [ivr])
                      - plsc.load_gather(py_v, [ivc]))
                dz = (plsc.load_gather(pz_v, [ivr])
                      - plsc.load_gather(pz_v, [ivc]))
                gx[0, osl] = dx
                gx[1, osl] = dy
                gx[2, osl] = dz
                gx[3, osl] = dx * dx + dy * dy + dz * dz

        def wait_gather(b):
            pltpu.make_async_copy(t1_hbm.at[ir_v.at[pl.ds(0, CH)]],
                                  r1s[b], sgs[b]).wait()
            pltpu.make_async_copy(t2_hbm.at[ic_v.at[pl.ds(0, CH)]],
                                  r2s[b], sgs[b]).wait()

        def start_out(k, b):
            base = base0 + k * CH
            pltpu.async_copy(r1s[b], g1_hbm.at[pl.ds(base, CH)], sos[b])
            pltpu.async_copy(r2s[b], g2_hbm.at[pl.ds(base, CH)], sos[b])
            for i, hbm in enumerate(geo_hbms):
                pltpu.async_copy(gxs[b].at[i], hbm.at[pl.ds(base, CH)], sos[b])

        def wait_out(b):
            pltpu.make_async_copy(r1s[b], g1_hbm.at[pl.ds(0, CH)], sos[b]).wait()
            pltpu.make_async_copy(r2s[b], g2_hbm.at[pl.ds(0, CH)], sos[b]).wait()
            for i, hbm in enumerate(geo_hbms):
                pltpu.make_async_copy(gxs[b].at[i], hbm.at[pl.ds(0, CH)],
                                      sos[b]).wait()

        start(0, 0)
        start(1, 1)

        def step(k, b):
            wait_gather(b)
            geom(k, b)
            start_out(k, b)

        def macro(i, carry):
            k = i * NB
            for b_idx in range(NB):
                k_b = k + b_idx
                b = b_idx
                step(k_b, b)
                nb = (b + 2) % NB
                pl.when(k_b >= 1)(lambda: wait_out(nb))
                start(k_b + 2, nb)
            return carry

        lax.fori_loop(0, (NCHUNK - 2) // NB, macro, 0)
        for k_b in (NCHUNK - 2, NCHUNK - 1):
            step(k_b, k_b % NB)
        for b in range(NB):
            wait_out(b)

    return body_fn


def _sc_gather(t1, t2, row, col, px, py, pz):
    return _gather_kernel()(t1, t2, row, col, px, py, pz)


def _edge_body(g1_ref, g2_ref, dx_ref, dy_ref, dz_ref, sq_ref,
               w256_ref, we2_ref, b2_ref, wc1_ref, bc1_ref, wc2_ref,
               m_ref, px_ref, py_ref, pz_ref):
    f = g1_ref[...] + g2_ref[...]
    sq = sq_ref[0].reshape(BE, 1)
    x1 = jax.nn.silu(f + sq * w256_ref[...])
    m = jax.nn.silu(jnp.dot(x1, we2_ref[...], preferred_element_type=jnp.float32)
                    + b2_ref[...])
    t = jax.nn.silu(jnp.dot(m, wc1_ref[...], preferred_element_type=jnp.float32)
                    + bc1_ref[...])
    cw = jnp.dot(t, wc2_ref[...], preferred_element_type=jnp.float32)
    scale = (cw * lax.rsqrt(sq + 1e-8)).reshape(1, 1, BE)
    m_ref[...] = m
    px_ref[...] = dx_ref[...] * scale
    py_ref[...] = dy_ref[...] * scale
    pz_ref[...] = dz_ref[...] * scale


def _edge(g1, g2, dxr, dyr, dzr, sqr, w256, we2, b2, wc1, bc1, wc2):
    row_spec = pl.BlockSpec((1, 1, BE), lambda i: (i, 0, 0))
    full = lambda shape: pl.BlockSpec(shape, lambda i: (0, 0))
    return pl.pallas_call(
        _edge_body,
        grid=(EB,),
        in_specs=[
            pl.BlockSpec((BE, D), lambda i: (i, 0)),
            pl.BlockSpec((BE, D), lambda i: (i, 0)),
            row_spec, row_spec, row_spec, row_spec,
            full((1, D)), full((D, D)), full((1, D)),
            full((D, D)), full((1, D)), full((D, 1)),
        ],
        out_specs=[
            pl.BlockSpec((BE, D), lambda i: (i, 0)),
            row_spec, row_spec, row_spec,
        ],
        out_shape=[jax.ShapeDtypeStruct((E, D), jnp.float32),
                   jax.ShapeDtypeStruct((EB, 1, BE), jnp.float32),
                   jax.ShapeDtypeStruct((EB, 1, BE), jnp.float32),
                   jax.ShapeDtypeStruct((EB, 1, BE), jnp.float32)],
    )(g1, g2, dxr, dyr, dzr, sqr, w256, we2, b2, wc1, bc1, wc2)


@functools.cache
def _scatter_kernel():
    @functools.partial(
        pl.kernel,
        out_type=jax.ShapeDtypeStruct((NC, NPAD, D), jnp.float32),
        mesh=_mesh(),
        scratch_types=[
            [pltpu.VMEM((CH,), jnp.int32)] * 2,
            [pltpu.VMEM((CH, D), jnp.float32)] * 2,
            [pltpu.SemaphoreType.DMA] * 2,
            pltpu.VMEM_SHARED((NPAD, D), jnp.float32),
        ],
    )
    def body_fn(m_hbm, row_hbm, z_hbm, pm_hbm, ivs, mbs, sms, accum):
        c = lax.axis_index("c")
        s = lax.axis_index("s")
        pltpu.sync_copy(z_hbm.at[pl.ds(s * RPT, RPT)],
                        accum.at[pl.ds(s * RPT, RPT)])
        plsc.subcore_barrier()
        base0 = (c * NS + s) * EW

        def start(k, b):
            base = base0 + k * CH
            pltpu.async_copy(row_hbm.at[pl.ds(base, CH)], ivs[b], sms[b])
            pltpu.async_copy(m_hbm.at[pl.ds(base, CH)], mbs[b], sms[b])

        def wait_in(b):
            pltpu.make_async_copy(row_hbm.at[pl.ds(0, CH)], ivs[b],
                                  sms[b]).wait()
            pltpu.make_async_copy(m_hbm.at[pl.ds(0, CH)], mbs[b],
                                  sms[b]).wait()

        start(0, 0)
        start(1, 1)

        def step(k, b):
            wait_in(b)
            pltpu.sync_copy(mbs[b], accum.at[ivs[b]], add=True)
            pl.when(k + 2 < NCHUNK)(lambda: start(k + 2, b))

        def macro(i, carry):
            k = i * 2
            step(k, 0)
            step(k + 1, 1)
            return carry

        lax.fori_loop(0, (NCHUNK - 1) // 2, macro, 0)
        step(NCHUNK - 1, (NCHUNK - 1) % 2)
        plsc.subcore_barrier()
        pltpu.sync_copy(accum.at[pl.ds(s * RPT, RPT)],
                        pm_hbm.at[c, pl.ds(s * RPT, RPT)])

    return body_fn


def _sc_scatter(m, row, zeros2d):
    return _scatter_kernel()(m, row, zeros2d)


@functools.cache
def _pos_scatter_kernel():
    @functools.partial(
        pl.kernel,
        out_type=jax.ShapeDtypeStruct((NW, P4), jnp.float32),
        mesh=_mesh(),
        compiler_params=pltpu.CompilerParams(needs_layout_passes=False),
        scratch_types=[
            [pltpu.VMEM((CHP,), jnp.int32)] * 2,
            [pltpu.VMEM((CHP,), jnp.float32)] * 2,
            [pltpu.VMEM((CHP,), jnp.float32)] * 2,
            [pltpu.VMEM((CHP,), jnp.float32)] * 2,
            [pltpu.SemaphoreType.DMA] * 2,
            pltpu.VMEM((P4,), jnp.float32),
        ],
    )
    def body_fn(row_hbm, pux_hbm, puy_hbm, puz_hbm, z4_hbm, pp_hbm,
                ivs, pxs, pys, pzs, sms, pacc):
        c = lax.axis_index("c")
        s = lax.axis_index("s")
        pltpu.sync_copy(z4_hbm, pacc)
        base0 = (c * NS + s) * EW
        ones = jnp.ones((L,), jnp.float32)

        def start(k, b):
            base = base0 + k * CHP
            pltpu.async_copy(row_hbm.at[pl.ds(base, CHP)], ivs[b], sms[b])
            pltpu.async_copy(pux_hbm.at[pl.ds(base, CHP)], pxs[b], sms[b])
            pltpu.async_copy(puy_hbm.at[pl.ds(base, CHP)], pys[b], sms[b])
            pltpu.async_copy(puz_hbm.at[pl.ds(base, CHP)], pzs[b], sms[b])

        def wait_in(b):
            pltpu.make_async_copy(row_hbm.at[pl.ds(0, CHP)], ivs[b],
                                  sms[b]).wait()
            for buf in (pxs[b], pys[b], pzs[b]):
                pltpu.make_async_copy(pux_hbm.at[pl.ds(0, CHP)], buf,
                                      sms[b]).wait()

        start(0, 0)
        start(1, 1)

        def step(k, b):
            wait_in(b)
            iv, pxb, pyb, pzb = ivs[b], pxs[b], pys[b], pzs[b]

            def group(j, carry):
                sl = pl.ds(j * L, L)
                i4 = iv[sl] * 4
                plsc.addupdate_scatter(pacc, [i4], pxb[sl])
                plsc.addupdate_scatter(pacc, [i4 + 1], pyb[sl])
                plsc.addupdate_scatter(pacc, [i4 + 2], pzb[sl])
                plsc.addupdate_scatter(pacc, [i4 + 3], ones)
                return carry

            lax.fori_loop(0, CHP // L, group, 0)
            pl.when(k + 2 < NCHP)(lambda: start(k + 2, b))

        def macro(i, carry):
            step(i * 2, 0)
            step(i * 2 + 1, 1)
            return carry

        lax.fori_loop(0, NCHP // 2, macro, 0)
        if NCHP % 2:
            step(NCHP - 1, (NCHP - 1) % 2)
        pltpu.sync_copy(pacc, pp_hbm.at[c * NS + s])

    return body_fn


def _sc_pos_scatter(row, pux, puy, puz, zeros4):
    return _pos_scatter_kernel()(row, pux, puy, puz, zeros4)


def _node_body(h_ref, pos_ref, pm0_ref, pm1_ref, pp_ref,
               wn1a_ref, wn1b_ref, bn1_ref, wn2_ref, bn2_ref, ho_ref, po_ref):
    h = h_ref[...]
    m_i = pm0_ref[...] + pm1_ref[...]
    q = jnp.sum(pp_ref[...], axis=0)
    num = q[:, :3]
    cnt = q[:, 3:4]
    x = (jnp.dot(h, wn1a_ref[...], preferred_element_type=jnp.float32)
         + jnp.dot(m_i, wn1b_ref[...], preferred_element_type=jnp.float32)
         + bn1_ref[...])
    hu = (jnp.dot(jax.nn.silu(x), wn2_ref[...], preferred_element_type=jnp.float32)
          + bn2_ref[...])
    ho_ref[...] = h + hu
    po_ref[...] = pos_ref[...] + num / jnp.maximum(cnt, 1.0)


def _node(h, pos, pm0, pm1, pp, wn1a, wn1b, bn1, wn2, bn2):
    return pl.pallas_call(
        _node_body,
        grid=(N // BN,),
        in_specs=[
            pl.BlockSpec((BN, D), lambda i: (i, 0)),
            pl.BlockSpec((BN, 3), lambda i: (i, 0)),
            pl.BlockSpec((BN, D), lambda i: (i, 0)),
            pl.BlockSpec((BN, D), lambda i: (i, 0)),
            pl.BlockSpec((NW, BN, 4), lambda i: (0, i, 0)),
            pl.BlockSpec((D, D), lambda i: (0, 0)),
            pl.BlockSpec((D, D), lambda i: (0, 0)),
            pl.BlockSpec((1, D), lambda i: (0, 0)),
            pl.BlockSpec((D, D), lambda i: (0, 0)),
            pl.BlockSpec((1, D), lambda i: (0, 0)),
        ],
        out_specs=[
            pl.BlockSpec((BN, D), lambda i: (i, 0)),
            pl.BlockSpec((BN, 3), lambda i: (i, 0)),
        ],
        out_shape=[jax.ShapeDtypeStruct((N, D), jnp.float32),
                   jax.ShapeDtypeStruct((N, 3), jnp.float32)],
    )(h, pos, pm0, pm1, pp, wn1a, wn1b, bn1, wn2, bn2)


def kernel(h, pos, edge_index, W_e1, b_e1, W_e2, b_e2, W_c1, b_c1, W_c2,
           W_n1, b_n1, W_n2, b_n2):
    row = edge_index[0].astype(jnp.int32)
    col = edge_index[1].astype(jnp.int32)
    px = pos[:, 0]
    py = pos[:, 1]
    pz = pos[:, 2]

    t1, t2 = _prep(h, W_e1[:D], W_e1[D:2 * D], b_e1.reshape(1, D))
    g1, g2, dxa, dya, dza, sqa = _sc_gather(t1, t2, row, col, px, py, pz)
    m, pux, puy, puz = _edge(
        g1, g2, dxa.reshape(EB, 1, BE), dya.reshape(EB, 1, BE),
        dza.reshape(EB, 1, BE), sqa.reshape(EB, 1, BE),
        W_e1[2 * D:2 * D + 1], W_e2, b_e2.reshape(1, D),
        W_c1, b_c1.reshape(1, D), W_c2)
    pm = _sc_scatter(m, row, jnp.zeros((NPAD, D), jnp.float32))
    pp = _sc_pos_scatter(row, pux.reshape(E), puy.reshape(E),
                         puz.reshape(E), jnp.zeros((P4,), jnp.float32))
    pp = pp.reshape(NW, NPAD, 4)
    h_out, pos_out = _node(h, pos, pm[0], pm[1], pp,
                           W_n1[:D], W_n1[D:], b_n1.reshape(1, D),
                           W_n2, b_n2.reshape(1, D))
    return h_out, pos_out


# speedup vs baseline: 5.6151x; 1.2397x over previous
import functools

import jax
import jax.numpy as jnp
from jax import lax
from jax.experimental import pallas as pl
from jax.experimental.pallas import tpu as pltpu
from jax.experimental.pallas import tpu_sc as plsc

N = 10000
E = 320000
D = 128
L = 16
NC, NS = 2, 16
NW = NC * NS
EW = E // NW
CH = 80
NG = CH // L
NCHUNK = EW // CH
CHP = 2000
NCHP = EW // CHP
NPAD = 10240
RPT = NPAD // NS
P4 = NPAD * 4
PPT = P4 // NS

BN = 400
BE = 2000
EB = E // BE


@functools.cache
def _mesh():
    return plsc.VectorSubcoreMesh(
        core_axis_name="c", subcore_axis_name="s",
        num_cores=NC, num_subcores=NS)


def _prep_body(h_ref, wa_ref, wb_ref, b1_ref, t1_ref, t2_ref):
    h = h_ref[...]
    t1_ref[...] = jnp.dot(h, wa_ref[...],
                          preferred_element_type=jnp.float32) + b1_ref[...]
    t2_ref[...] = jnp.dot(h, wb_ref[...], preferred_element_type=jnp.float32)


def _prep(h, wa, wb, b1):
    return pl.pallas_call(
        _prep_body,
        grid=(N // BN,),
        in_specs=[
            pl.BlockSpec((BN, D), lambda i: (i, 0)),
            pl.BlockSpec((D, D), lambda i: (0, 0)),
            pl.BlockSpec((D, D), lambda i: (0, 0)),
            pl.BlockSpec((1, D), lambda i: (0, 0)),
        ],
        out_specs=[
            pl.BlockSpec((BN, D), lambda i: (i, 0)),
            pl.BlockSpec((BN, D), lambda i: (i, 0)),
        ],
        out_shape=[jax.ShapeDtypeStruct((N, D), jnp.float32)] * 2,
    )(h, wa, wb, b1)


NB = 3


@functools.cache
def _gather_kernel():
    @functools.partial(
        pl.kernel,
        out_type=[jax.ShapeDtypeStruct((E, D), jnp.float32),
                  jax.ShapeDtypeStruct((E, D), jnp.float32),
                  jax.ShapeDtypeStruct((E,), jnp.float32),
                  jax.ShapeDtypeStruct((E,), jnp.float32),
                  jax.ShapeDtypeStruct((E,), jnp.float32),
                  jax.ShapeDtypeStruct((E,), jnp.float32)],
        mesh=_mesh(),
        compiler_params=pltpu.CompilerParams(needs_layout_passes=False),
        scratch_types=[
            pltpu.VMEM((EW,), jnp.int32),
            pltpu.VMEM((EW,), jnp.int32),
            pltpu.VMEM((N,), jnp.float32),
            pltpu.VMEM((N,), jnp.float32),
            pltpu.VMEM((N,), jnp.float32),
            [pltpu.VMEM((CH, D), jnp.float32)] * NB,
            [pltpu.VMEM((CH, D), jnp.float32)] * NB,
            [pltpu.VMEM((4, CH), jnp.float32)] * NB,
            [pltpu.SemaphoreType.DMA] * NB,
            [pltpu.SemaphoreType.DMA] * NB,
        ],
    )
    def body_fn(t1_hbm, t2_hbm, row_hbm, col_hbm, px_hbm, py_hbm, pz_hbm,
                g1_hbm, g2_hbm, dx_hbm, dy_hbm, dz_hbm, sq_hbm,
                ir_v, ic_v, px_v, py_v, pz_v, r1s, r2s, gxs, sgs, sos):
        geo_hbms = (dx_hbm, dy_hbm, dz_hbm, sq_hbm)
        wid = lax.axis_index("s") * NC + lax.axis_index("c")
        base0 = wid * EW
        pltpu.sync_copy(row_hbm.at[pl.ds(base0, EW)], ir_v)
        pltpu.sync_copy(col_hbm.at[pl.ds(base0, EW)], ic_v)
        pltpu.sync_copy(px_hbm, px_v)
        pltpu.sync_copy(py_hbm, py_v)
        pltpu.sync_copy(pz_hbm, pz_v)

        def start(k, b):
            off = k * CH
            pltpu.async_copy(t1_hbm.at[ir_v.at[pl.ds(off, CH)]], r1s[b], sgs[b])
            pltpu.async_copy(t2_hbm.at[ic_v.at[pl.ds(off, CH)]], r2s[b], sgs[b])

        def geom(k, b):
            gx = gxs[b]
            for j in range(NG):
                sl = pl.ds(k * CH + j * L, L)
                osl = pl.ds(j * L, L)
                ivr = ir_v[sl]
                ivc = ic_v[sl]
                dx = (plsc.load_gather(px_v, [ivr])
                      - plsc.load_gather(px_v, [ivc]))
                dy = (plsc.load_gather(py_v, [ivr])
                      - plsc.load_gather(py_v, [ivc]))
                dz = (plsc.load_gather(pz_v, [ivr])
                      - plsc.load_gather(pz_v, [ivc]))
                gx[0, osl] = dx
                gx[1, osl] = dy
                gx[2, osl] = dz
                gx[3, osl] = dx * dx + dy * dy + dz * dz

        def wait_gather(b):
            pltpu.make_async_copy(t1_hbm.at[ir_v.at[pl.ds(0, CH)]],
                                  r1s[b], sgs[b]).wait()
            pltpu.make_async_copy(t2_hbm.at[ic_v.at[pl.ds(0, CH)]],
                                  r2s[b], sgs[b]).wait()

        def start_out(k, b):
            base = base0 + k * CH
            pltpu.async_copy(r1s[b], g1_hbm.at[pl.ds(base, CH)], sos[b])
            pltpu.async_copy(r2s[b], g2_hbm.at[pl.ds(base, CH)], sos[b])
            for i, hbm in enumerate(geo_hbms):
                pltpu.async_copy(gxs[b].at[i], hbm.at[pl.ds(base, CH)], sos[b])

        def wait_out(b):
            pltpu.make_async_copy(r1s[b], g1_hbm.at[pl.ds(0, CH)], sos[b]).wait()
            pltpu.make_async_copy(r2s[b], g2_hbm.at[pl.ds(0, CH)], sos[b]).wait()
            for i, hbm in enumerate(geo_hbms):
                pltpu.make_async_copy(gxs[b].at[i], hbm.at[pl.ds(0, CH)],
                                      sos[b]).wait()

        start(0, 0)
        start(1, 1)

        def step(k, b):
            wait_gather(b)
            geom(k, b)
            start_out(k, b)

        def macro(i, carry):
            k = i * NB
            for b_idx in range(NB):
                k_b = k + b_idx
                b = b_idx
                step(k_b, b)
                nb = (b + 2) % NB
                pl.when(k_b >= 1)(lambda: wait_out(nb))
                start(k_b + 2, nb)
            return carry

        lax.fori_loop(0, (NCHUNK - 2) // NB, macro, 0)
        for k_b in (NCHUNK - 2, NCHUNK - 1):
            step(k_b, k_b % NB)
        for b in range(NB):
            wait_out(b)

    return body_fn


def _sc_gather(t1, t2, row, col, px, py, pz):
    return _gather_kernel()(t1, t2, row, col, px, py, pz)


def _edge_body(g1_ref, g2_ref, dx_ref, dy_ref, dz_ref, sq_ref,
               w256_ref, we2_ref, b2_ref, wc1_ref, bc1_ref, wc2_ref,
               m_ref, px_ref, py_ref, pz_ref):
    f = g1_ref[...] + g2_ref[...]
    sq = sq_ref[0].reshape(BE, 1)
    x1 = jax.nn.silu(f + sq * w256_ref[...])
    m = jax.nn.silu(jnp.dot(x1, we2_ref[...], preferred_element_type=jnp.float32)
                    + b2_ref[...])
    t = jax.nn.silu(jnp.dot(m, wc1_ref[...], preferred_element_type=jnp.float32)
                    + bc1_ref[...])
    cw = jnp.dot(t, wc2_ref[...], preferred_element_type=jnp.float32)
    scale = (cw * lax.rsqrt(sq + 1e-8)).reshape(1, 1, BE)
    m_ref[...] = m
    px_ref[...] = dx_ref[...] * scale
    py_ref[...] = dy_ref[...] * scale
    pz_ref[...] = dz_ref[...] * scale


def _edge(g1, g2, dxr, dyr, dzr, sqr, w256, we2, b2, wc1, bc1, wc2):
    row_spec = pl.BlockSpec((1, 1, BE), lambda i: (i, 0, 0))
    full = lambda shape: pl.BlockSpec(shape, lambda i: (0, 0))
    return pl.pallas_call(
        _edge_body,
        grid=(EB,),
        in_specs=[
            pl.BlockSpec((BE, D), lambda i: (i, 0)),
            pl.BlockSpec((BE, D), lambda i: (i, 0)),
            row_spec, row_spec, row_spec, row_spec,
            full((1, D)), full((D, D)), full((1, D)),
            full((D, D)), full((1, D)), full((D, 1)),
        ],
        out_specs=[
            pl.BlockSpec((BE, D), lambda i: (i, 0)),
            row_spec, row_spec, row_spec,
        ],
        out_shape=[jax.ShapeDtypeStruct((E, D), jnp.float32),
                   jax.ShapeDtypeStruct((EB, 1, BE), jnp.float32),
                   jax.ShapeDtypeStruct((EB, 1, BE), jnp.float32),
                   jax.ShapeDtypeStruct((EB, 1, BE), jnp.float32)],
    )(g1, g2, dxr, dyr, dzr, sqr, w256, we2, b2, wc1, bc1, wc2)


@functools.cache
def _scatter_kernel():
    @functools.partial(
        pl.kernel,
        out_type=jax.ShapeDtypeStruct((NC, NPAD, D), jnp.float32),
        mesh=_mesh(),
        scratch_types=[
            [pltpu.VMEM((CH,), jnp.int32)] * 2,
            [pltpu.VMEM((CH, D), jnp.float32)] * 2,
            [pltpu.SemaphoreType.DMA] * 2,
            pltpu.VMEM_SHARED((NPAD, D), jnp.float32),
        ],
    )
    def body_fn(m_hbm, row_hbm, z_hbm, pm_hbm, ivs, mbs, sms, accum):
        c = lax.axis_index("c")
        s = lax.axis_index("s")
        pltpu.sync_copy(z_hbm.at[pl.ds(s * RPT, RPT)],
                        accum.at[pl.ds(s * RPT, RPT)])
        plsc.subcore_barrier()
        base0 = (c * NS + s) * EW

        def start(k, b):
            base = base0 + k * CH
            pltpu.async_copy(row_hbm.at[pl.ds(base, CH)], ivs[b], sms[b])
            pltpu.async_copy(m_hbm.at[pl.ds(base, CH)], mbs[b], sms[b])

        def wait_in(b):
            pltpu.make_async_copy(row_hbm.at[pl.ds(0, CH)], ivs[b],
                                  sms[b]).wait()
            pltpu.make_async_copy(m_hbm.at[pl.ds(0, CH)], mbs[b],
                                  sms[b]).wait()

        start(0, 0)
        start(1, 1)

        def step(k, b):
            wait_in(b)
            pltpu.sync_copy(mbs[b], accum.at[ivs[b]], add=True)
            pl.when(k + 2 < NCHUNK)(lambda: start(k + 2, b))

        def macro(i, carry):
            k = i * 2
            step(k, 0)
            step(k + 1, 1)
            return carry

        lax.fori_loop(0, (NCHUNK - 1) // 2, macro, 0)
        step(NCHUNK - 1, (NCHUNK - 1) % 2)
        plsc.subcore_barrier()
        pltpu.sync_copy(accum.at[pl.ds(s * RPT, RPT)],
                        pm_hbm.at[c, pl.ds(s * RPT, RPT)])

    return body_fn


def _sc_scatter(m, row, zeros2d):
    return _scatter_kernel()(m, row, zeros2d)


@functools.cache
def _pos_scatter_kernel():
    @functools.partial(
        pl.kernel,
        out_type=jax.ShapeDtypeStruct((NW, P4), jnp.float32),
        mesh=_mesh(),
        compiler_params=pltpu.CompilerParams(needs_layout_passes=False),
        scratch_types=[
            [pltpu.VMEM((CHP,), jnp.int32)] * 2,
            [pltpu.VMEM((CHP,), jnp.float32)] * 2,
            [pltpu.VMEM((CHP,), jnp.float32)] * 2,
            [pltpu.VMEM((CHP,), jnp.float32)] * 2,
            [pltpu.SemaphoreType.DMA] * 2,
            pltpu.VMEM((P4,), jnp.float32),
        ],
    )
    def body_fn(row_hbm, pux_hbm, puy_hbm, puz_hbm, z4_hbm, pp_hbm,
                ivs, pxs, pys, pzs, sms, pacc):
        c = lax.axis_index("c")
        s = lax.axis_index("s")
        pltpu.sync_copy(z4_hbm, pacc)
        base0 = (c * NS + s) * EW
        ones = jnp.ones((L,), jnp.float32)

        def start(k, b):
            base = base0 + k * CHP
            pltpu.async_copy(row_hbm.at[pl.ds(base, CHP)], ivs[b], sms[b])
            pltpu.async_copy(pux_hbm.at[pl.ds(base, CHP)], pxs[b], sms[b])
            pltpu.async_copy(puy_hbm.at[pl.ds(base, CHP)], pys[b], sms[b])
            pltpu.async_copy(puz_hbm.at[pl.ds(base, CHP)], pzs[b], sms[b])

        def wait_in(b):
            pltpu.make_async_copy(row_hbm.at[pl.ds(0, CHP)], ivs[b],
                                  sms[b]).wait()
            for buf in (pxs[b], pys[b], pzs[b]):
                pltpu.make_async_copy(pux_hbm.at[pl.ds(0, CHP)], buf,
                                      sms[b]).wait()

        start(0, 0)
        start(1, 1)

        def step(k, b):
            wait_in(b)
            iv, pxb, pyb, pzb = ivs[b], pxs[b], pys[b], pzs[b]

            def group(j, carry):
                sl = pl.ds(j * L, L)
                i4 = iv[sl] * 4
                plsc.addupdate_scatter(pacc, [i4], pxb[sl])
                plsc.addupdate_scatter(pacc, [i4 + 1], pyb[sl])
                plsc.addupdate_scatter(pacc, [i4 + 2], pzb[sl])
                plsc.addupdate_scatter(pacc, [i4 + 3], ones)
                return carry

            lax.fori_loop(0, CHP // L, group, 0)
            pl.when(k + 2 < NCHP)(lambda: start(k + 2, b))

        def macro(i, carry):
            step(i * 2, 0)
            step(i * 2 + 1, 1)
            return carry

        lax.fori_loop(0, NCHP // 2, macro, 0)
        if NCHP % 2:
            step(NCHP - 1, (NCHP - 1) % 2)
        pltpu.sync_copy(pacc, pp_hbm.at[c * NS + s])

    return body_fn


def _sc_pos_scatter(row, pux, puy, puz, zeros4):
    return _pos_scatter_kernel()(row, pux, puy, puz, zeros4)


def _node_body(h_ref, pos_ref, pm0_ref, pm1_ref, pp_ref,
               wn1a_ref, wn1b_ref, bn1_ref, wn2_ref, bn2_ref, ho_ref, po_ref):
    h = h_ref[...]
    m_i = pm0_ref[...] + pm1_ref[...]
    q = jnp.sum(pp_ref[...], axis=0)
    num = q[:, :3]
    cnt = q[:, 3:4]
    x = (jnp.dot(h, wn1a_ref[...], preferred_element_type=jnp.float32)
         + jnp.dot(m_i, wn1b_ref[...], preferred_element_type=jnp.float32)
         + bn1_ref[...])
    hu = (jnp.dot(jax.nn.silu(x), wn2_ref[...], preferred_element_type=jnp.float32)
          + bn2_ref[...])
    ho_ref[...] = h + hu
    po_ref[...] = pos_ref[...] + num / jnp.maximum(cnt, 1.0)


def _node(h, pos, pm0, pm1, pp, wn1a, wn1b, bn1, wn2, bn2):
    return pl.pallas_call(
        _node_body,
        grid=(N // BN,),
        in_specs=[
            pl.BlockSpec((BN, D), lambda i: (i, 0)),
            pl.BlockSpec((BN, 3), lambda i: (i, 0)),
            pl.BlockSpec((BN, D), lambda i: (i, 0)),
            pl.BlockSpec((BN, D), lambda i: (i, 0)),
            pl.BlockSpec((NW, BN, 4), lambda i: (0, i, 0)),
            pl.BlockSpec((D, D), lambda i: (0, 0)),
            pl.BlockSpec((D, D), lambda i: (0, 0)),
            pl.BlockSpec((1, D), lambda i: (0, 0)),
            pl.BlockSpec((D, D), lambda i: (0, 0)),
            pl.BlockSpec((1, D), lambda i: (0, 0)),
        ],
        out_specs=[
            pl.BlockSpec((BN, D), lambda i: (i, 0)),
            pl.BlockSpec((BN, 3), lambda i: (i, 0)),
        ],
        out_shape=[jax.ShapeDtypeStruct((N, D), jnp.float32),
                   jax.ShapeDtypeStruct((N, 3), jnp.float32)],
    )(h, pos, pm0, pm1, pp, wn1a, wn1b, bn1, wn2, bn2)


def kernel(h, pos, edge_index, W_e1, b_e1, W_e2, b_e2, W_c1, b_c1, W_c2,
           W_n1, b_n1, W_n2, b_n2):
    row = edge_index[0].astype(jnp.int32)
    col = edge_index[1].astype(jnp.int32)
    px = pos[:, 0]
    py = pos[:, 1]
    pz = pos[:, 2]

    t1, t2 = _prep(h, W_e1[:D], W_e1[D:2 * D], b_e1.reshape(1, D))
    g1, g2, dxa, dya, dza, sqa = _sc_gather(t1, t2, row, col, px, py, pz)
    m, pux, puy, puz = _edge(
        g1, g2, dxa.reshape(EB, 1, BE), dya.reshape(EB, 1, BE),
        dza.reshape(EB, 1, BE), sqa.reshape(EB, 1, BE),
        W_e1[2 * D:2 * D + 1], W_e2, b_e2.reshape(1, D),
        W_c1, b_c1.reshape(1, D), W_c2)
    pm = _sc_scatter(m, row, jnp.zeros((NPAD, D), jnp.float32))
    pp = _sc_pos_scatter(row, pux.reshape(E), puy.reshape(E),
                         puz.reshape(E), jnp.zeros((P4,), jnp.float32))
    pp = pp.reshape(NW, NPAD, 4)
    h_out, pos_out = _node(h, pos, pm[0], pm[1], pp,
                           W_n1[:D], W_n1[D:], b_n1.reshape(1, D),
                           W_n2, b_n2.reshape(1, D))
    return h_out, pos_out


# speedup vs baseline: 6.4550x; 1.1496x over previous
import functools

import jax
import jax.numpy as jnp
from jax import lax
from jax.experimental import pallas as pl
from jax.experimental.pallas import tpu as pltpu
from jax.experimental.pallas import tpu_sc as plsc

N = 10000
E = 320000
D = 128
L = 16
NC, NS = 2, 16
NW = NC * NS
EW = E // NW
CH = 80
NG = CH // L
NCHUNK = EW // CH
CHP = 2000
NCHP = EW // CHP
NPAD = 10240
RPT = NPAD // NS
P4 = NPAD * 4
PPT = P4 // NS

BN = 400
BE = 2000
EB = E // BE


@functools.cache
def _mesh():
    return plsc.VectorSubcoreMesh(
        core_axis_name="c", subcore_axis_name="s",
        num_cores=NC, num_subcores=NS)


def _prep_body(h_ref, wa_ref, wb_ref, b1_ref, t1_ref, t2_ref):
    h = h_ref[...]
    t1_ref[...] = jnp.dot(h, wa_ref[...],
                          preferred_element_type=jnp.float32) + b1_ref[...]
    t2_ref[...] = jnp.dot(h, wb_ref[...], preferred_element_type=jnp.float32)


def _prep(h, wa, wb, b1):
    return pl.pallas_call(
        _prep_body,
        grid=(N // BN,),
        in_specs=[
            pl.BlockSpec((BN, D), lambda i: (i, 0)),
            pl.BlockSpec((D, D), lambda i: (0, 0)),
            pl.BlockSpec((D, D), lambda i: (0, 0)),
            pl.BlockSpec((1, D), lambda i: (0, 0)),
        ],
        out_specs=[
            pl.BlockSpec((BN, D), lambda i: (i, 0)),
            pl.BlockSpec((BN, D), lambda i: (i, 0)),
        ],
        out_shape=[jax.ShapeDtypeStruct((N, D), jnp.float32)] * 2,
    )(h, wa, wb, b1)


NB = 3


@functools.cache
def _gather_kernel():
    @functools.partial(
        pl.kernel,
        out_type=[jax.ShapeDtypeStruct((E, D), jnp.float32),
                  jax.ShapeDtypeStruct((E, D), jnp.float32),
                  jax.ShapeDtypeStruct((E,), jnp.float32),
                  jax.ShapeDtypeStruct((E,), jnp.float32),
                  jax.ShapeDtypeStruct((E,), jnp.float32),
                  jax.ShapeDtypeStruct((E,), jnp.float32)],
        mesh=_mesh(),
        compiler_params=pltpu.CompilerParams(needs_layout_passes=False),
        scratch_types=[
            pltpu.VMEM((EW,), jnp.int32),
            pltpu.VMEM((EW,), jnp.int32),
            pltpu.VMEM((N,), jnp.float32),
            pltpu.VMEM((N,), jnp.float32),
            pltpu.VMEM((N,), jnp.float32),
            [pltpu.VMEM((CH, D), jnp.float32)] * NB,
            [pltpu.VMEM((CH, D), jnp.float32)] * NB,
            [pltpu.VMEM((4, CH), jnp.float32)] * NB,
            [pltpu.SemaphoreType.DMA] * NB,
            [pltpu.SemaphoreType.DMA] * NB,
        ],
    )
    def body_fn(t1_hbm, t2_hbm, row_hbm, col_hbm, px_hbm, py_hbm, pz_hbm,
                g1_hbm, g2_hbm, dx_hbm, dy_hbm, dz_hbm, sq_hbm,
                ir_v, ic_v, px_v, py_v, pz_v, r1s, r2s, gxs, sgs, sos):
        geo_hbms = (dx_hbm, dy_hbm, dz_hbm, sq_hbm)
        wid = lax.axis_index("s") * NC + lax.axis_index("c")
        base0 = wid * EW
        pltpu.sync_copy(row_hbm.at[pl.ds(base0, EW)], ir_v)
        pltpu.sync_copy(col_hbm.at[pl.ds(base0, EW)], ic_v)
        pltpu.sync_copy(px_hbm, px_v)
        pltpu.sync_copy(py_hbm, py_v)
        pltpu.sync_copy(pz_hbm, pz_v)

        def start(k, b):
            off = k * CH
            pltpu.async_copy(t1_hbm.at[ir_v.at[pl.ds(off, CH)]], r1s[b], sgs[b])
            pltpu.async_copy(t2_hbm.at[ic_v.at[pl.ds(off, CH)]], r2s[b], sgs[b])

        def geom(k, b):
            gx = gxs[b]
            for j in range(NG):
                sl = pl.ds(k * CH + j * L, L)
                osl = pl.ds(j * L, L)
                ivr = ir_v[sl]
                ivc = ic_v[sl]
                dx = (plsc.load_gather(px_v, [ivr])
                      - plsc.load_gather(px_v, [ivc]))
                dy = (plsc.load_gather(py_v, [ivr])
                      - plsc.load_gather(py_v, [ivc]))
                dz = (plsc.load_gather(pz_v, [ivr])
                      - plsc.load_gather(pz_v, [ivc]))
                gx[0, osl] = dx
                gx[1, osl] = dy
                gx[2, osl] = dz
                gx[3, osl] = dx * dx + dy * dy + dz * dz

        def wait_gather(b):
            pltpu.make_async_copy(t1_hbm.at[ir_v.at[pl.ds(0, CH)]],
                                  r1s[b], sgs[b]).wait()
            pltpu.make_async_copy(t2_hbm.at[ic_v.at[pl.ds(0, CH)]],
                                  r2s[b], sgs[b]).wait()

        def start_out(k, b):
            base = base0 + k * CH
            pltpu.async_copy(r1s[b], g1_hbm.at[pl.ds(base, CH)], sos[b])
            pltpu.async_copy(r2s[b], g2_hbm.at[pl.ds(base, CH)], sos[b])
            for i, hbm in enumerate(geo_hbms):
                pltpu.async_copy(gxs[b].at[i], hbm.at[pl.ds(base, CH)], sos[b])

        def wait_out(b):
            pltpu.make_async_copy(r1s[b], g1_hbm.at[pl.ds(0, CH)], sos[b]).wait()
            pltpu.make_async_copy(r2s[b], g2_hbm.at[pl.ds(0, CH)], sos[b]).wait()
            for i, hbm in enumerate(geo_hbms):
                pltpu.make_async_copy(gxs[b].at[i], hbm.at[pl.ds(0, CH)],
                                      sos[b]).wait()

        start(0, 0)
        start(1, 1)

        def step(k, b):
            wait_gather(b)
            geom(k, b)
            start_out(k, b)

        def macro(i, carry):
            k = i * NB
            for b_idx in range(NB):
                k_b = k + b_idx
                b = b_idx
                step(k_b, b)
                nb = (b + 2) % NB
                pl.when(k_b >= 1)(lambda: wait_out(nb))
                start(k_b + 2, nb)
            return carry

        lax.fori_loop(0, (NCHUNK - 2) // NB, macro, 0)
        for k_b in (NCHUNK - 2, NCHUNK - 1):
            step(k_b, k_b % NB)
        for b in range(NB):
            wait_out(b)

    return body_fn


def _sc_gather(t1, t2, row, col, px, py, pz):
    return _gather_kernel()(t1, t2, row, col, px, py, pz)


def _edge_body(g1_ref, g2_ref, dx_ref, dy_ref, dz_ref, sq_ref,
               w256_ref, we2_ref, b2_ref, wc1_ref, bc1_ref, wc2_ref,
               m_ref, px_ref, py_ref, pz_ref):
    f = g1_ref[...] + g2_ref[...]
    sq = sq_ref[0].T
    x1 = jax.nn.silu(f + sq * w256_ref[...])
    m = jax.nn.silu(jnp.dot(x1, we2_ref[...], preferred_element_type=jnp.float32)
                    + b2_ref[...])
    t = jax.nn.silu(jnp.dot(m, wc1_ref[...], preferred_element_type=jnp.float32)
                    + bc1_ref[...])
    cw = jnp.dot(t, wc2_ref[...], preferred_element_type=jnp.float32)
    scale = (cw * lax.rsqrt(sq + 1e-8)).T.reshape(1, 1, BE)
    m_ref[...] = m
    px_ref[...] = dx_ref[...] * scale
    py_ref[...] = dy_ref[...] * scale
    pz_ref[...] = dz_ref[...] * scale


def _edge(g1, g2, dxr, dyr, dzr, sqr, w256, we2, b2, wc1, bc1, wc2):
    row_spec = pl.BlockSpec((1, 1, BE), lambda i: (i, 0, 0))
    full = lambda shape: pl.BlockSpec(shape, lambda i: (0, 0))
    return pl.pallas_call(
        _edge_body,
        grid=(EB,),
        in_specs=[
            pl.BlockSpec((BE, D), lambda i: (i, 0)),
            pl.BlockSpec((BE, D), lambda i: (i, 0)),
            row_spec, row_spec, row_spec, row_spec,
            full((1, D)), full((D, D)), full((1, D)),
            full((D, D)), full((1, D)), full((D, 1)),
        ],
        out_specs=[
            pl.BlockSpec((BE, D), lambda i: (i, 0)),
            row_spec, row_spec, row_spec,
        ],
        out_shape=[jax.ShapeDtypeStruct((E, D), jnp.float32),
                   jax.ShapeDtypeStruct((EB, 1, BE), jnp.float32),
                   jax.ShapeDtypeStruct((EB, 1, BE), jnp.float32),
                   jax.ShapeDtypeStruct((EB, 1, BE), jnp.float32)],
    )(g1, g2, dxr, dyr, dzr, sqr, w256, we2, b2, wc1, bc1, wc2)


@functools.cache
def _scatter_kernel():
    @functools.partial(
        pl.kernel,
        out_type=jax.ShapeDtypeStruct((NC, NPAD, D), jnp.float32),
        mesh=_mesh(),
        scratch_types=[
            [pltpu.VMEM((CH,), jnp.int32)] * 2,
            [pltpu.VMEM((CH, D), jnp.float32)] * 2,
            [pltpu.SemaphoreType.DMA] * 2,
            pltpu.VMEM_SHARED((NPAD, D), jnp.float32),
        ],
    )
    def body_fn(m_hbm, row_hbm, z_hbm, pm_hbm, ivs, mbs, sms, accum):
        c = lax.axis_index("c")
        s = lax.axis_index("s")
        pltpu.sync_copy(z_hbm.at[pl.ds(s * RPT, RPT)],
                        accum.at[pl.ds(s * RPT, RPT)])
        plsc.subcore_barrier()
        base0 = (c * NS + s) * EW

        def start(k, b):
            base = base0 + k * CH
            pltpu.async_copy(row_hbm.at[pl.ds(base, CH)], ivs[b], sms[b])
            pltpu.async_copy(m_hbm.at[pl.ds(base, CH)], mbs[b], sms[b])

        def wait_in(b):
            pltpu.make_async_copy(row_hbm.at[pl.ds(0, CH)], ivs[b],
                                  sms[b]).wait()
            pltpu.make_async_copy(m_hbm.at[pl.ds(0, CH)], mbs[b],
                                  sms[b]).wait()

        start(0, 0)
        start(1, 1)

        def step(k, b):
            wait_in(b)
            pltpu.sync_copy(mbs[b], accum.at[ivs[b]], add=True)
            pl.when(k + 2 < NCHUNK)(lambda: start(k + 2, b))

        def macro(i, carry):
            k = i * 2
            step(k, 0)
            step(k + 1, 1)
            return carry

        lax.fori_loop(0, (NCHUNK - 1) // 2, macro, 0)
        step(NCHUNK - 1, (NCHUNK - 1) % 2)
        plsc.subcore_barrier()
        pltpu.sync_copy(accum.at[pl.ds(s * RPT, RPT)],
                        pm_hbm.at[c, pl.ds(s * RPT, RPT)])

    return body_fn


def _sc_scatter(m, row, zeros2d):
    return _scatter_kernel()(m, row, zeros2d)


@functools.cache
def _pos_scatter_kernel():
    @functools.partial(
        pl.kernel,
        out_type=jax.ShapeDtypeStruct((NW, P4), jnp.float32),
        mesh=_mesh(),
        compiler_params=pltpu.CompilerParams(needs_layout_passes=False),
        scratch_types=[
            [pltpu.VMEM((CHP,), jnp.int32)] * 2,
            [pltpu.VMEM((CHP,), jnp.float32)] * 2,
            [pltpu.VMEM((CHP,), jnp.float32)] * 2,
            [pltpu.VMEM((CHP,), jnp.float32)] * 2,
            [pltpu.SemaphoreType.DMA] * 2,
            pltpu.VMEM((P4,), jnp.float32),
        ],
    )
    def body_fn(row_hbm, pux_hbm, puy_hbm, puz_hbm, z4_hbm, pp_hbm,
                ivs, pxs, pys, pzs, sms, pacc):
        c = lax.axis_index("c")
        s = lax.axis_index("s")
        pltpu.sync_copy(z4_hbm, pacc)
        base0 = (c * NS + s) * EW
        ones = jnp.ones((L,), jnp.float32)

        def start(k, b):
            base = base0 + k * CHP
            pltpu.async_copy(row_hbm.at[pl.ds(base, CHP)], ivs[b], sms[b])
            pltpu.async_copy(pux_hbm.at[pl.ds(base, CHP)], pxs[b], sms[b])
            pltpu.async_copy(puy_hbm.at[pl.ds(base, CHP)], pys[b], sms[b])
            pltpu.async_copy(puz_hbm.at[pl.ds(base, CHP)], pzs[b], sms[b])

        def wait_in(b):
            pltpu.make_async_copy(row_hbm.at[pl.ds(0, CHP)], ivs[b],
                                  sms[b]).wait()
            for buf in (pxs[b], pys[b], pzs[b]):
                pltpu.make_async_copy(pux_hbm.at[pl.ds(0, CHP)], buf,
                                      sms[b]).wait()

        start(0, 0)
        start(1, 1)

        def step(k, b):
            wait_in(b)
            iv, pxb, pyb, pzb = ivs[b], pxs[b], pys[b], pzs[b]

            def group(j, carry):
                sl = pl.ds(j * L, L)
                i4 = iv[sl] * 4
                plsc.addupdate_scatter(pacc, [i4], pxb[sl])
                plsc.addupdate_scatter(pacc, [i4 + 1], pyb[sl])
                plsc.addupdate_scatter(pacc, [i4 + 2], pzb[sl])
                plsc.addupdate_scatter(pacc, [i4 + 3], ones)
                return carry

            lax.fori_loop(0, CHP // L, group, 0)
            pl.when(k + 2 < NCHP)(lambda: start(k + 2, b))

        def macro(i, carry):
            step(i * 2, 0)
            step(i * 2 + 1, 1)
            return carry

        lax.fori_loop(0, NCHP // 2, macro, 0)
        if NCHP % 2:
            step(NCHP - 1, (NCHP - 1) % 2)
        pltpu.sync_copy(pacc, pp_hbm.at[c * NS + s])

    return body_fn


def _sc_pos_scatter(row, pux, puy, puz, zeros4):
    return _pos_scatter_kernel()(row, pux, puy, puz, zeros4)


def _node_body(h_ref, pos_ref, pm0_ref, pm1_ref, pp_ref,
               wn1a_ref, wn1b_ref, bn1_ref, wn2_ref, bn2_ref, ho_ref, po_ref):
    h = h_ref[...]
    m_i = pm0_ref[...] + pm1_ref[...]
    q = jnp.sum(pp_ref[...], axis=0)
    num = q[:, :3]
    cnt = q[:, 3:4]
    x = (jnp.dot(h, wn1a_ref[...], preferred_element_type=jnp.float32)
         + jnp.dot(m_i, wn1b_ref[...], preferred_element_type=jnp.float32)
         + bn1_ref[...])
    hu = (jnp.dot(jax.nn.silu(x), wn2_ref[...], preferred_element_type=jnp.float32)
          + bn2_ref[...])
    ho_ref[...] = h + hu
    po_ref[...] = pos_ref[...] + num / jnp.maximum(cnt, 1.0)


def _node(h, pos, pm0, pm1, pp, wn1a, wn1b, bn1, wn2, bn2):
    return pl.pallas_call(
        _node_body,
        grid=(N // BN,),
        in_specs=[
            pl.BlockSpec((BN, D), lambda i: (i, 0)),
            pl.BlockSpec((BN, 3), lambda i: (i, 0)),
            pl.BlockSpec((BN, D), lambda i: (i, 0)),
            pl.BlockSpec((BN, D), lambda i: (i, 0)),
            pl.BlockSpec((NW, BN, 4), lambda i: (0, i, 0)),
            pl.BlockSpec((D, D), lambda i: (0, 0)),
            pl.BlockSpec((D, D), lambda i: (0, 0)),
            pl.BlockSpec((1, D), lambda i: (0, 0)),
            pl.BlockSpec((D, D), lambda i: (0, 0)),
            pl.BlockSpec((1, D), lambda i: (0, 0)),
        ],
        out_specs=[
            pl.BlockSpec((BN, D), lambda i: (i, 0)),
            pl.BlockSpec((BN, 3), lambda i: (i, 0)),
        ],
        out_shape=[jax.ShapeDtypeStruct((N, D), jnp.float32),
                   jax.ShapeDtypeStruct((N, 3), jnp.float32)],
    )(h, pos, pm0, pm1, pp, wn1a, wn1b, bn1, wn2, bn2)


def kernel(h, pos, edge_index, W_e1, b_e1, W_e2, b_e2, W_c1, b_c1, W_c2,
           W_n1, b_n1, W_n2, b_n2):
    row = edge_index[0].astype(jnp.int32)
    col = edge_index[1].astype(jnp.int32)
    px = pos[:, 0]
    py = pos[:, 1]
    pz = pos[:, 2]

    t1, t2 = _prep(h, W_e1[:D], W_e1[D:2 * D], b_e1.reshape(1, D))
    g1, g2, dxa, dya, dza, sqa = _sc_gather(t1, t2, row, col, px, py, pz)
    m, pux, puy, puz = _edge(
        g1, g2, dxa.reshape(EB, 1, BE), dya.reshape(EB, 1, BE),
        dza.reshape(EB, 1, BE), sqa.reshape(EB, 1, BE),
        W_e1[2 * D:2 * D + 1], W_e2, b_e2.reshape(1, D),
        W_c1, b_c1.reshape(1, D), W_c2)
    pm = _sc_scatter(m, row, jnp.zeros((NPAD, D), jnp.float32))
    pp = _sc_pos_scatter(row, pux.reshape(E), puy.reshape(E),
                         puz.reshape(E), jnp.zeros((P4,), jnp.float32))
    pp = pp.reshape(NW, NPAD, 4)
    h_out, pos_out = _node(h, pos, pm[0], pm[1], pp,
                           W_n1[:D], W_n1[D:], b_n1.reshape(1, D),
                           W_n2, b_n2.reshape(1, D))
    return h_out, pos_out


# speedup vs baseline: 6.9089x; 1.0703x over previous
import functools

import jax
import jax.numpy as jnp
from jax import lax
from jax.experimental import pallas as pl
from jax.experimental.pallas import tpu as pltpu
from jax.experimental.pallas import tpu_sc as plsc

N = 10000
E = 320000
D = 128
L = 16
NC, NS = 2, 16
NW = NC * NS
EW = E // NW
CH = 80
NG = CH // L
NCHUNK = EW // CH
CHP = 2000
NCHP = EW // CHP
NPAD = 10240
RPT = NPAD // NS
P4 = NPAD * 4
PPT = P4 // NS

BN = 400
BE = 2000
EB = E // BE


@functools.cache
def _mesh():
    return plsc.VectorSubcoreMesh(
        core_axis_name="c", subcore_axis_name="s",
        num_cores=NC, num_subcores=NS)


def _prep_body(h_ref, wa_ref, wb_ref, b1_ref, t1_ref, t2_ref):
    h = h_ref[...]
    t1_ref[...] = jnp.dot(h, wa_ref[...],
                          preferred_element_type=jnp.float32) + b1_ref[...]
    t2_ref[...] = jnp.dot(h, wb_ref[...], preferred_element_type=jnp.float32)


def _prep(h, wa, wb, b1):
    return pl.pallas_call(
        _prep_body,
        grid=(N // BN,),
        in_specs=[
            pl.BlockSpec((BN, D), lambda i: (i, 0)),
            pl.BlockSpec((D, D), lambda i: (0, 0)),
            pl.BlockSpec((D, D), lambda i: (0, 0)),
            pl.BlockSpec((1, D), lambda i: (0, 0)),
        ],
        out_specs=[
            pl.BlockSpec((BN, D), lambda i: (i, 0)),
            pl.BlockSpec((BN, D), lambda i: (i, 0)),
        ],
        out_shape=[jax.ShapeDtypeStruct((N, D), jnp.float32)] * 2,
    )(h, wa, wb, b1)


NB = 3


@functools.cache
def _gather_kernel():
    @functools.partial(
        pl.kernel,
        out_type=[jax.ShapeDtypeStruct((E, D), jnp.float32),
                  jax.ShapeDtypeStruct((E,), jnp.float32),
                  jax.ShapeDtypeStruct((E,), jnp.float32),
                  jax.ShapeDtypeStruct((E,), jnp.float32),
                  jax.ShapeDtypeStruct((E,), jnp.float32)],
        mesh=_mesh(),
        compiler_params=pltpu.CompilerParams(needs_layout_passes=False),
        scratch_types=[
            pltpu.VMEM((EW,), jnp.int32),
            pltpu.VMEM((EW,), jnp.int32),
            pltpu.VMEM((N,), jnp.float32),
            pltpu.VMEM((N,), jnp.float32),
            pltpu.VMEM((N,), jnp.float32),
            [pltpu.VMEM((CH, D), jnp.float32)] * NB,
            [pltpu.VMEM((CH, D), jnp.float32)] * NB,
            [pltpu.VMEM((4, CH), jnp.float32)] * NB,
            [pltpu.SemaphoreType.DMA] * NB,
            [pltpu.SemaphoreType.DMA] * NB,
        ],
    )
    def body_fn(t1_hbm, t2_hbm, row_hbm, col_hbm, px_hbm, py_hbm, pz_hbm,
                g_hbm, dx_hbm, dy_hbm, dz_hbm, sq_hbm,
                ir_v, ic_v, px_v, py_v, pz_v, r1s, r2s, gxs, sgs, sos):
        geo_hbms = (dx_hbm, dy_hbm, dz_hbm, sq_hbm)
        wid = lax.axis_index("s") * NC + lax.axis_index("c")
        base0 = wid * EW
        pltpu.sync_copy(row_hbm.at[pl.ds(base0, EW)], ir_v)
        pltpu.sync_copy(col_hbm.at[pl.ds(base0, EW)], ic_v)
        pltpu.sync_copy(px_hbm, px_v)
        pltpu.sync_copy(py_hbm, py_v)
        pltpu.sync_copy(pz_hbm, pz_v)

        def start(k, b):
            off = k * CH
            pltpu.async_copy(t1_hbm.at[ir_v.at[pl.ds(off, CH)]], r1s[b], sgs[b])
            pltpu.async_copy(t2_hbm.at[ic_v.at[pl.ds(off, CH)]], r2s[b], sgs[b])

        def geom(k, b):
            gx = gxs[b]
            for j in range(NG):
                sl = pl.ds(k * CH + j * L, L)
                osl = pl.ds(j * L, L)
                ivr = ir_v[sl]
                ivc = ic_v[sl]
                dx = (plsc.load_gather(px_v, [ivr])
                      - plsc.load_gather(px_v, [ivc]))
                dy = (plsc.load_gather(py_v, [ivr])
                      - plsc.load_gather(py_v, [ivc]))
                dz = (plsc.load_gather(pz_v, [ivr])
                      - plsc.load_gather(pz_v, [ivc]))
                gx[0, osl] = dx
                gx[1, osl] = dy
                gx[2, osl] = dz
                gx[3, osl] = dx * dx + dy * dy + dz * dz

        def wait_gather(b):
            pltpu.make_async_copy(t1_hbm.at[ir_v.at[pl.ds(0, CH)]],
                                  r1s[b], sgs[b]).wait()
            pltpu.make_async_copy(t2_hbm.at[ic_v.at[pl.ds(0, CH)]],
                                  r2s[b], sgs[b]).wait()

        def accum_rows(b):
            r1, r2 = r1s[b], r2s[b]

            def erow(e, carry):
                for d in range(D // L):
                    sl = pl.ds(d * L, L)
                    plsc.addupdate(r1.at[e, sl], r2[e, sl])
                return carry

            lax.fori_loop(0, CH, erow, 0)

        def start_out(k, b):
            base = base0 + k * CH
            pltpu.async_copy(r1s[b], g_hbm.at[pl.ds(base, CH)], sos[b])
            for i, hbm in enumerate(geo_hbms):
                pltpu.async_copy(gxs[b].at[i], hbm.at[pl.ds(base, CH)], sos[b])

        def wait_out(b):
            pltpu.make_async_copy(r1s[b], g_hbm.at[pl.ds(0, CH)], sos[b]).wait()
            for i, hbm in enumerate(geo_hbms):
                pltpu.make_async_copy(gxs[b].at[i], hbm.at[pl.ds(0, CH)],
                                      sos[b]).wait()

        start(0, 0)
        start(1, 1)

        def step(k, b):
            wait_gather(b)
            geom(k, b)
            accum_rows(b)
            start_out(k, b)

        def macro(i, carry):
            k = i * NB
            for b_idx in range(NB):
                k_b = k + b_idx
                b = b_idx
                step(k_b, b)
                nb = (b + 2) % NB
                pl.when(k_b >= 1)(lambda: wait_out(nb))
                start(k_b + 2, nb)
            return carry

        lax.fori_loop(0, (NCHUNK - 2) // NB, macro, 0)
        for k_b in (NCHUNK - 2, NCHUNK - 1):
            step(k_b, k_b % NB)
        for b in range(NB):
            wait_out(b)

    return body_fn


def _sc_gather(t1, t2, row, col, px, py, pz):
    return _gather_kernel()(t1, t2, row, col, px, py, pz)


def _edge_body(g_ref, dx_ref, dy_ref, dz_ref, sq_ref,
               w256_ref, we2_ref, b2_ref, wc1_ref, bc1_ref, wc2_ref,
               m_ref, px_ref, py_ref, pz_ref):
    f = g_ref[...]
    sq = sq_ref[0].T
    x1 = jax.nn.silu(f + sq * w256_ref[...])
    m = jax.nn.silu(jnp.dot(x1, we2_ref[...], preferred_element_type=jnp.float32)
                    + b2_ref[...])
    t = jax.nn.silu(jnp.dot(m, wc1_ref[...], preferred_element_type=jnp.float32)
                    + bc1_ref[...])
    cw = jnp.dot(t, wc2_ref[...], preferred_element_type=jnp.float32)
    scale = (cw * lax.rsqrt(sq + 1e-8)).T.reshape(1, 1, BE)
    m_ref[...] = m
    px_ref[...] = dx_ref[...] * scale
    py_ref[...] = dy_ref[...] * scale
    pz_ref[...] = dz_ref[...] * scale


def _edge(g, dxr, dyr, dzr, sqr, w256, we2, b2, wc1, bc1, wc2):
    row_spec = pl.BlockSpec((1, 1, BE), lambda i: (i, 0, 0))
    full = lambda shape: pl.BlockSpec(shape, lambda i: (0, 0))
    return pl.pallas_call(
        _edge_body,
        grid=(EB,),
        in_specs=[
            pl.BlockSpec((BE, D), lambda i: (i, 0)),
            row_spec, row_spec, row_spec, row_spec,
            full((1, D)), full((D, D)), full((1, D)),
            full((D, D)), full((1, D)), full((D, 1)),
        ],
        out_specs=[
            pl.BlockSpec((BE, D), lambda i: (i, 0)),
            row_spec, row_spec, row_spec,
        ],
        out_shape=[jax.ShapeDtypeStruct((E, D), jnp.float32),
                   jax.ShapeDtypeStruct((EB, 1, BE), jnp.float32),
                   jax.ShapeDtypeStruct((EB, 1, BE), jnp.float32),
                   jax.ShapeDtypeStruct((EB, 1, BE), jnp.float32)],
    )(g, dxr, dyr, dzr, sqr, w256, we2, b2, wc1, bc1, wc2)


@functools.cache
def _scatter_kernel():
    @functools.partial(
        pl.kernel,
        out_type=jax.ShapeDtypeStruct((NC, NPAD, D), jnp.float32),
        mesh=_mesh(),
        scratch_types=[
            [pltpu.VMEM((CH,), jnp.int32)] * 2,
            [pltpu.VMEM((CH, D), jnp.float32)] * 2,
            [pltpu.SemaphoreType.DMA] * 2,
            pltpu.VMEM_SHARED((NPAD, D), jnp.float32),
        ],
    )
    def body_fn(m_hbm, row_hbm, z_hbm, pm_hbm, ivs, mbs, sms, accum):
        c = lax.axis_index("c")
        s = lax.axis_index("s")
        pltpu.sync_copy(z_hbm.at[pl.ds(s * RPT, RPT)],
                        accum.at[pl.ds(s * RPT, RPT)])
        plsc.subcore_barrier()
        base0 = (c * NS + s) * EW

        def start(k, b):
            base = base0 + k * CH
            pltpu.async_copy(row_hbm.at[pl.ds(base, CH)], ivs[b], sms[b])
            pltpu.async_copy(m_hbm.at[pl.ds(base, CH)], mbs[b], sms[b])

        def wait_in(b):
            pltpu.make_async_copy(row_hbm.at[pl.ds(0, CH)], ivs[b],
                                  sms[b]).wait()
            pltpu.make_async_copy(m_hbm.at[pl.ds(0, CH)], mbs[b],
                                  sms[b]).wait()

        start(0, 0)
        start(1, 1)

        def step(k, b):
            wait_in(b)
            pltpu.sync_copy(mbs[b], accum.at[ivs[b]], add=True)
            pl.when(k + 2 < NCHUNK)(lambda: start(k + 2, b))

        def macro(i, carry):
            k = i * 2
            step(k, 0)
            step(k + 1, 1)
            return carry

        lax.fori_loop(0, (NCHUNK - 1) // 2, macro, 0)
        step(NCHUNK - 1, (NCHUNK - 1) % 2)
        plsc.subcore_barrier()
        pltpu.sync_copy(accum.at[pl.ds(s * RPT, RPT)],
                        pm_hbm.at[c, pl.ds(s * RPT, RPT)])

    return body_fn


def _sc_scatter(m, row, zeros2d):
    return _scatter_kernel()(m, row, zeros2d)


@functools.cache
def _pos_scatter_kernel():
    @functools.partial(
        pl.kernel,
        out_type=jax.ShapeDtypeStruct((NW, P4), jnp.float32),
        mesh=_mesh(),
        compiler_params=pltpu.CompilerParams(needs_layout_passes=False),
        scratch_types=[
            [pltpu.VMEM((CHP,), jnp.int32)] * 2,
            [pltpu.VMEM((CHP,), jnp.float32)] * 2,
            [pltpu.VMEM((CHP,), jnp.float32)] * 2,
            [pltpu.VMEM((CHP,), jnp.float32)] * 2,
            [pltpu.SemaphoreType.DMA] * 2,
            pltpu.VMEM((P4,), jnp.float32),
        ],
    )
    def body_fn(row_hbm, pux_hbm, puy_hbm, puz_hbm, z4_hbm, pp_hbm,
                ivs, pxs, pys, pzs, sms, pacc):
        c = lax.axis_index("c")
        s = lax.axis_index("s")
        pltpu.sync_copy(z4_hbm, pacc)
        base0 = (c * NS + s) * EW
        ones = jnp.ones((L,), jnp.float32)

        def start(k, b):
            base = base0 + k * CHP
            pltpu.async_copy(row_hbm.at[pl.ds(base, CHP)], ivs[b], sms[b])
            pltpu.async_copy(pux_hbm.at[pl.ds(base, CHP)], pxs[b], sms[b])
            pltpu.async_copy(puy_hbm.at[pl.ds(base, CHP)], pys[b], sms[b])
            pltpu.async_copy(puz_hbm.at[pl.ds(base, CHP)], pzs[b], sms[b])

        def wait_in(b):
            pltpu.make_async_copy(row_hbm.at[pl.ds(0, CHP)], ivs[b],
                                  sms[b]).wait()
            for buf in (pxs[b], pys[b], pzs[b]):
                pltpu.make_async_copy(pux_hbm.at[pl.ds(0, CHP)], buf,
                                      sms[b]).wait()

        start(0, 0)
        start(1, 1)

        def step(k, b):
            wait_in(b)
            iv, pxb, pyb, pzb = ivs[b], pxs[b], pys[b], pzs[b]

            def group(j, carry):
                sl = pl.ds(j * L, L)
                i4 = iv[sl] * 4
                plsc.addupdate_scatter(pacc, [i4], pxb[sl])
                plsc.addupdate_scatter(pacc, [i4 + 1], pyb[sl])
                plsc.addupdate_scatter(pacc, [i4 + 2], pzb[sl])
                plsc.addupdate_scatter(pacc, [i4 + 3], ones)
                return carry

            lax.fori_loop(0, CHP // L, group, 0)
            pl.when(k + 2 < NCHP)(lambda: start(k + 2, b))

        def macro(i, carry):
            step(i * 2, 0)
            step(i * 2 + 1, 1)
            return carry

        lax.fori_loop(0, NCHP // 2, macro, 0)
        if NCHP % 2:
            step(NCHP - 1, (NCHP - 1) % 2)
        pltpu.sync_copy(pacc, pp_hbm.at[c * NS + s])

    return body_fn


def _sc_pos_scatter(row, pux, puy, puz, zeros4):
    return _pos_scatter_kernel()(row, pux, puy, puz, zeros4)


def _node_body(h_ref, pos_ref, pm0_ref, pm1_ref, pp_ref,
               wn1a_ref, wn1b_ref, bn1_ref, wn2_ref, bn2_ref, ho_ref, po_ref):
    h = h_ref[...]
    m_i = pm0_ref[...] + pm1_ref[...]
    q = jnp.sum(pp_ref[...], axis=0)
    num = q[:, :3]
    cnt = q[:, 3:4]
    x = (jnp.dot(h, wn1a_ref[...], preferred_element_type=jnp.float32)
         + jnp.dot(m_i, wn1b_ref[...], preferred_element_type=jnp.float32)
         + bn1_ref[...])
    hu = (jnp.dot(jax.nn.silu(x), wn2_ref[...], preferred_element_type=jnp.float32)
          + bn2_ref[...])
    ho_ref[...] = h + hu
    po_ref[...] = pos_ref[...] + num / jnp.maximum(cnt, 1.0)


def _node(h, pos, pm0, pm1, pp, wn1a, wn1b, bn1, wn2, bn2):
    return pl.pallas_call(
        _node_body,
        grid=(N // BN,),
        in_specs=[
            pl.BlockSpec((BN, D), lambda i: (i, 0)),
            pl.BlockSpec((BN, 3), lambda i: (i, 0)),
            pl.BlockSpec((BN, D), lambda i: (i, 0)),
            pl.BlockSpec((BN, D), lambda i: (i, 0)),
            pl.BlockSpec((NW, BN, 4), lambda i: (0, i, 0)),
            pl.BlockSpec((D, D), lambda i: (0, 0)),
            pl.BlockSpec((D, D), lambda i: (0, 0)),
            pl.BlockSpec((1, D), lambda i: (0, 0)),
            pl.BlockSpec((D, D), lambda i: (0, 0)),
            pl.BlockSpec((1, D), lambda i: (0, 0)),
        ],
        out_specs=[
            pl.BlockSpec((BN, D), lambda i: (i, 0)),
            pl.BlockSpec((BN, 3), lambda i: (i, 0)),
        ],
        out_shape=[jax.ShapeDtypeStruct((N, D), jnp.float32),
                   jax.ShapeDtypeStruct((N, 3), jnp.float32)],
    )(h, pos, pm0, pm1, pp, wn1a, wn1b, bn1, wn2, bn2)


def kernel(h, pos, edge_index, W_e1, b_e1, W_e2, b_e2, W_c1, b_c1, W_c2,
           W_n1, b_n1, W_n2, b_n2):
    row = edge_index[0].astype(jnp.int32)
    col = edge_index[1].astype(jnp.int32)
    px = pos[:, 0]
    py = pos[:, 1]
    pz = pos[:, 2]

    t1, t2 = _prep(h, W_e1[:D], W_e1[D:2 * D], b_e1.reshape(1, D))
    g, dxa, dya, dza, sqa = _sc_gather(t1, t2, row, col, px, py, pz)
    m, pux, puy, puz = _edge(
        g, dxa.reshape(EB, 1, BE), dya.reshape(EB, 1, BE),
        dza.reshape(EB, 1, BE), sqa.reshape(EB, 1, BE),
        W_e1[2 * D:2 * D + 1], W_e2, b_e2.reshape(1, D),
        W_c1, b_c1.reshape(1, D), W_c2)
    pm = _sc_scatter(m, row, jnp.zeros((NPAD, D), jnp.float32))
    pp = _sc_pos_scatter(row, pux.reshape(E), puy.reshape(E),
                         puz.reshape(E), jnp.zeros((P4,), jnp.float32))
    pp = pp.reshape(NW, NPAD, 4)
    h_out, pos_out = _node(h, pos, pm[0], pm[1], pp,
                           W_n1[:D], W_n1[D:], b_n1.reshape(1, D),
                           W_n2, b_n2.reshape(1, D))
    return h_out, pos_out


# speedup vs baseline: 7.3556x; 1.0646x over previous
import functools

import jax
import jax.numpy as jnp
from jax import lax
from jax.experimental import pallas as pl
from jax.experimental.pallas import tpu as pltpu
from jax.experimental.pallas import tpu_sc as plsc

N = 10000
E = 320000
D = 128
L = 16
NC, NS = 2, 16
NW = NC * NS
EW = E // NW
CH = 80
NG = CH // L
NCHUNK = EW // CH
CHP = 2000
NCHP = EW // CHP
NPAD = 10240
RPT = NPAD // NS
P4 = NPAD * 4
PPT = P4 // NS

BN = 400
BE = 2000
EB = E // BE


@functools.cache
def _mesh():
    return plsc.VectorSubcoreMesh(
        core_axis_name="c", subcore_axis_name="s",
        num_cores=NC, num_subcores=NS)


def _prep_body(h_ref, wa_ref, wb_ref, b1_ref, t1_ref, t2_ref):
    h = h_ref[...]
    t1_ref[...] = jnp.dot(h, wa_ref[...],
                          preferred_element_type=jnp.float32) + b1_ref[...]
    t2_ref[...] = jnp.dot(h, wb_ref[...], preferred_element_type=jnp.float32)


def _prep(h, wa, wb, b1):
    return pl.pallas_call(
        _prep_body,
        grid=(N // BN,),
        in_specs=[
            pl.BlockSpec((BN, D), lambda i: (i, 0)),
            pl.BlockSpec((D, D), lambda i: (0, 0)),
            pl.BlockSpec((D, D), lambda i: (0, 0)),
            pl.BlockSpec((1, D), lambda i: (0, 0)),
        ],
        out_specs=[
            pl.BlockSpec((BN, D), lambda i: (i, 0)),
            pl.BlockSpec((BN, D), lambda i: (i, 0)),
        ],
        out_shape=[jax.ShapeDtypeStruct((N, D), jnp.float32)] * 2,
    )(h, wa, wb, b1)


NB = 3


@functools.cache
def _gather_kernel():
    @functools.partial(
        pl.kernel,
        out_type=[jax.ShapeDtypeStruct((E, D), jnp.float32),
                  jax.ShapeDtypeStruct((E,), jnp.float32),
                  jax.ShapeDtypeStruct((E,), jnp.float32),
                  jax.ShapeDtypeStruct((E,), jnp.float32),
                  jax.ShapeDtypeStruct((E,), jnp.float32)],
        mesh=_mesh(),
        compiler_params=pltpu.CompilerParams(needs_layout_passes=False),
        scratch_types=[
            pltpu.VMEM((EW,), jnp.int32),
            pltpu.VMEM((EW,), jnp.int32),
            pltpu.VMEM((N,), jnp.float32),
            pltpu.VMEM((N,), jnp.float32),
            pltpu.VMEM((N,), jnp.float32),
            [pltpu.VMEM((CH, D), jnp.float32)] * NB,
            [pltpu.VMEM((CH, D), jnp.float32)] * NB,
            [pltpu.VMEM((4, CH), jnp.float32)] * NB,
            [pltpu.SemaphoreType.DMA] * NB,
            [pltpu.SemaphoreType.DMA] * NB,
        ],
    )
    def body_fn(t1_hbm, t2_hbm, row_hbm, col_hbm, px_hbm, py_hbm, pz_hbm,
                g_hbm, dx_hbm, dy_hbm, dz_hbm, sq_hbm,
                ir_v, ic_v, px_v, py_v, pz_v, r1s, r2s, gxs, sgs, sos):
        geo_hbms = (dx_hbm, dy_hbm, dz_hbm, sq_hbm)
        wid = lax.axis_index("s") * NC + lax.axis_index("c")
        base0 = wid * EW
        pltpu.sync_copy(row_hbm.at[pl.ds(base0, EW)], ir_v)
        pltpu.sync_copy(col_hbm.at[pl.ds(base0, EW)], ic_v)
        pltpu.sync_copy(px_hbm, px_v)
        pltpu.sync_copy(py_hbm, py_v)
        pltpu.sync_copy(pz_hbm, pz_v)

        def start(k, b):
            off = k * CH
            pltpu.async_copy(t1_hbm.at[ir_v.at[pl.ds(off, CH)]], r1s[b], sgs[b])
            pltpu.async_copy(t2_hbm.at[ic_v.at[pl.ds(off, CH)]], r2s[b], sgs[b])

        def geom(k, b):
            gx = gxs[b]
            for j in range(NG):
                sl = pl.ds(k * CH + j * L, L)
                osl = pl.ds(j * L, L)
                ivr = ir_v[sl]
                ivc = ic_v[sl]
                dx = (plsc.load_gather(px_v, [ivr])
                      - plsc.load_gather(px_v, [ivc]))
                dy = (plsc.load_gather(py_v, [ivr])
                      - plsc.load_gather(py_v, [ivc]))
                dz = (plsc.load_gather(pz_v, [ivr])
                      - plsc.load_gather(pz_v, [ivc]))
                gx[0, osl] = dx
                gx[1, osl] = dy
                gx[2, osl] = dz
                gx[3, osl] = dx * dx + dy * dy + dz * dz

        def wait_gather(b):
            pltpu.make_async_copy(t1_hbm.at[ir_v.at[pl.ds(0, CH)]],
                                  r1s[b], sgs[b]).wait()
            pltpu.make_async_copy(t2_hbm.at[ic_v.at[pl.ds(0, CH)]],
                                  r2s[b], sgs[b]).wait()

        def accum_rows(b):
            r1, r2 = r1s[b], r2s[b]

            def erow(e, carry):
                for d in range(D // L):
                    sl = pl.ds(d * L, L)
                    plsc.addupdate(r1.at[e, sl], r2[e, sl])
                return carry

            lax.fori_loop(0, CH, erow, 0)

        def start_out(k, b):
            base = base0 + k * CH
            pltpu.async_copy(r1s[b], g_hbm.at[pl.ds(base, CH)], sos[b])
            for i, hbm in enumerate(geo_hbms):
                pltpu.async_copy(gxs[b].at[i], hbm.at[pl.ds(base, CH)], sos[b])

        def wait_out(b):
            pltpu.make_async_copy(r1s[b], g_hbm.at[pl.ds(0, CH)], sos[b]).wait()
            for i, hbm in enumerate(geo_hbms):
                pltpu.make_async_copy(gxs[b].at[i], hbm.at[pl.ds(0, CH)],
                                      sos[b]).wait()

        start(0, 0)
        start(1, 1)

        def step(k, b):
            wait_gather(b)
            geom(k, b)
            accum_rows(b)
            start_out(k, b)

        def macro(i, carry):
            k = i * NB
            for b_idx in range(NB):
                k_b = k + b_idx
                b = b_idx
                step(k_b, b)
                nb = (b + 2) % NB
                pl.when(k_b >= 1)(lambda: wait_out(nb))
                start(k_b + 2, nb)
            return carry

        lax.fori_loop(0, (NCHUNK - 2) // NB, macro, 0)
        for k_b in (NCHUNK - 2, NCHUNK - 1):
            step(k_b, k_b % NB)
        for b in range(NB):
            wait_out(b)

    return body_fn


def _sc_gather(t1, t2, row, col, px, py, pz):
    return _gather_kernel()(t1, t2, row, col, px, py, pz)


def _edge_body(g_ref, dx_ref, dy_ref, dz_ref, sq_ref,
               w256_ref, we2_ref, b2_ref, wc1_ref, bc1_ref, wc2_ref,
               m_ref, px_ref, py_ref, pz_ref):
    f = g_ref[...]
    sq = sq_ref[0].T
    x1 = jax.nn.silu(f + sq * w256_ref[...])
    m = jax.nn.silu(jnp.dot(x1, we2_ref[...], preferred_element_type=jnp.float32)
                    + b2_ref[...])
    t = jax.nn.silu(jnp.dot(m, wc1_ref[...], preferred_element_type=jnp.float32)
                    + bc1_ref[...])
    cw = jnp.dot(t, wc2_ref[...], preferred_element_type=jnp.float32)
    scale = (cw * lax.rsqrt(sq + 1e-8)).T.reshape(1, 1, BE)
    m_ref[...] = m
    px_ref[...] = dx_ref[...] * scale
    py_ref[...] = dy_ref[...] * scale
    pz_ref[...] = dz_ref[...] * scale


def _edge(g, dxr, dyr, dzr, sqr, w256, we2, b2, wc1, bc1, wc2):
    row_spec = pl.BlockSpec((1, 1, BE), lambda i: (i, 0, 0))
    full = lambda shape: pl.BlockSpec(shape, lambda i: (0, 0))
    return pl.pallas_call(
        _edge_body,
        grid=(EB,),
        in_specs=[
            pl.BlockSpec((BE, D), lambda i: (i, 0)),
            row_spec, row_spec, row_spec, row_spec,
            full((1, D)), full((D, D)), full((1, D)),
            full((D, D)), full((1, D)), full((D, 1)),
        ],
        out_specs=[
            pl.BlockSpec((BE, D), lambda i: (i, 0)),
            row_spec, row_spec, row_spec,
        ],
        out_shape=[jax.ShapeDtypeStruct((E, D), jnp.float32),
                   jax.ShapeDtypeStruct((EB, 1, BE), jnp.float32),
                   jax.ShapeDtypeStruct((EB, 1, BE), jnp.float32),
                   jax.ShapeDtypeStruct((EB, 1, BE), jnp.float32)],
    )(g, dxr, dyr, dzr, sqr, w256, we2, b2, wc1, bc1, wc2)


@functools.cache
def _scatter_kernel():
    @functools.partial(
        pl.kernel,
        out_type=jax.ShapeDtypeStruct((NC, NPAD, D), jnp.float32),
        mesh=_mesh(),
        scratch_types=[
            [pltpu.VMEM((CH,), jnp.int32)] * 2,
            [pltpu.VMEM((CH, D), jnp.float32)] * 2,
            [pltpu.SemaphoreType.DMA] * 2,
            pltpu.VMEM_SHARED((NPAD, D), jnp.float32),
        ],
    )
    def body_fn(m_hbm, row_hbm, z_hbm, pm_hbm, ivs, mbs, sms, accum):
        c = lax.axis_index("c")
        s = lax.axis_index("s")
        pltpu.sync_copy(z_hbm.at[pl.ds(s * RPT, RPT)],
                        accum.at[pl.ds(s * RPT, RPT)])
        plsc.subcore_barrier()
        base0 = (c * NS + s) * EW

        def start(k, b):
            base = base0 + k * CH
            pltpu.async_copy(row_hbm.at[pl.ds(base, CH)], ivs[b], sms[b])
            pltpu.async_copy(m_hbm.at[pl.ds(base, CH)], mbs[b], sms[b])

        def wait_in(b):
            pltpu.make_async_copy(row_hbm.at[pl.ds(0, CH)], ivs[b],
                                  sms[b]).wait()
            pltpu.make_async_copy(m_hbm.at[pl.ds(0, CH)], mbs[b],
                                  sms[b]).wait()

        start(0, 0)
        start(1, 1)

        def step(k, b):
            wait_in(b)
            pltpu.sync_copy(mbs[b], accum.at[ivs[b]], add=True)
            pl.when(k + 2 < NCHUNK)(lambda: start(k + 2, b))

        def macro(i, carry):
            k = i * 2
            step(k, 0)
            step(k + 1, 1)
            return carry

        lax.fori_loop(0, (NCHUNK - 1) // 2, macro, 0)
        step(NCHUNK - 1, (NCHUNK - 1) % 2)
        plsc.subcore_barrier()
        pltpu.sync_copy(accum.at[pl.ds(s * RPT, RPT)],
                        pm_hbm.at[c, pl.ds(s * RPT, RPT)])

    return body_fn


def _sc_scatter(m, row, zeros2d):
    return _scatter_kernel()(m, row, zeros2d)


@functools.cache
def _pos_scatter_kernel():
    @functools.partial(
        pl.kernel,
        out_type=jax.ShapeDtypeStruct((NC, P4), jnp.float32),
        mesh=_mesh(),
        compiler_params=pltpu.CompilerParams(needs_layout_passes=False),
        scratch_types=[
            [pltpu.VMEM((CHP,), jnp.int32)] * 2,
            [pltpu.VMEM((CHP,), jnp.float32)] * 2,
            [pltpu.VMEM((CHP,), jnp.float32)] * 2,
            [pltpu.VMEM((CHP,), jnp.float32)] * 2,
            [pltpu.SemaphoreType.DMA] * 2,
            pltpu.VMEM((P4,), jnp.float32),
            pltpu.VMEM((PPT,), jnp.float32),
            pltpu.VMEM((PPT,), jnp.float32),
            pltpu.VMEM_SHARED((NS, P4), jnp.float32),
        ],
    )
    def body_fn(row_hbm, pux_hbm, puy_hbm, puz_hbm, z4_hbm, pp_hbm,
                ivs, pxs, pys, pzs, sms, pacc, mbuf, tbuf, pstage):
        c = lax.axis_index("c")
        s = lax.axis_index("s")
        pltpu.sync_copy(z4_hbm, pacc)
        base0 = (c * NS + s) * EW
        ones = jnp.ones((L,), jnp.float32)

        def start(k, b):
            base = base0 + k * CHP
            pltpu.async_copy(row_hbm.at[pl.ds(base, CHP)], ivs[b], sms[b])
            pltpu.async_copy(pux_hbm.at[pl.ds(base, CHP)], pxs[b], sms[b])
            pltpu.async_copy(puy_hbm.at[pl.ds(base, CHP)], pys[b], sms[b])
            pltpu.async_copy(puz_hbm.at[pl.ds(base, CHP)], pzs[b], sms[b])

        def wait_in(b):
            pltpu.make_async_copy(row_hbm.at[pl.ds(0, CHP)], ivs[b],
                                  sms[b]).wait()
            for buf in (pxs[b], pys[b], pzs[b]):
                pltpu.make_async_copy(pux_hbm.at[pl.ds(0, CHP)], buf,
                                      sms[b]).wait()

        start(0, 0)
        start(1, 1)

        def step(k, b):
            wait_in(b)
            iv, pxb, pyb, pzb = ivs[b], pxs[b], pys[b], pzs[b]

            def group(j, carry):
                sl = pl.ds(j * L, L)
                i4 = iv[sl] * 4
                plsc.addupdate_scatter(pacc, [i4], pxb[sl])
                plsc.addupdate_scatter(pacc, [i4 + 1], pyb[sl])
                plsc.addupdate_scatter(pacc, [i4 + 2], pzb[sl])
                plsc.addupdate_scatter(pacc, [i4 + 3], ones)
                return carry

            lax.fori_loop(0, CHP // L, group, 0)
            pl.when(k + 2 < NCHP)(lambda: start(k + 2, b))

        def macro(i, carry):
            step(i * 2, 0)
            step(i * 2 + 1, 1)
            return carry

        lax.fori_loop(0, NCHP // 2, macro, 0)
        if NCHP % 2:
            step(NCHP - 1, (NCHP - 1) % 2)
        pltpu.sync_copy(pacc, pstage.at[s])
        plsc.subcore_barrier()
        pltpu.sync_copy(pstage.at[0, pl.ds(s * PPT, PPT)], mbuf)

        def merge(t, carry):
            pltpu.sync_copy(pstage.at[t, pl.ds(s * PPT, PPT)], tbuf)

            def add16(j, carry2):
                sl = pl.ds(j * L, L)
                plsc.addupdate(mbuf.at[sl], tbuf[sl])
                return carry2

            lax.fori_loop(0, PPT // L, add16, 0)
            return carry

        lax.fori_loop(1, NS, merge, 0)
        pltpu.sync_copy(mbuf, pp_hbm.at[c, pl.ds(s * PPT, PPT)])

    return body_fn


def _sc_pos_scatter(row, pux, puy, puz, zeros4):
    return _pos_scatter_kernel()(row, pux, puy, puz, zeros4)


def _node_body(h_ref, pos_ref, pm0_ref, pm1_ref, pp0_ref, pp1_ref,
               wn1a_ref, wn1b_ref, bn1_ref, wn2_ref, bn2_ref, ho_ref, po_ref):
    h = h_ref[...]
    m_i = pm0_ref[...] + pm1_ref[...]
    q = pp0_ref[...] + pp1_ref[...]
    num = q[:, :3]
    cnt = q[:, 3:4]
    x = (jnp.dot(h, wn1a_ref[...], preferred_element_type=jnp.float32)
         + jnp.dot(m_i, wn1b_ref[...], preferred_element_type=jnp.float32)
         + bn1_ref[...])
    hu = (jnp.dot(jax.nn.silu(x), wn2_ref[...], preferred_element_type=jnp.float32)
          + bn2_ref[...])
    ho_ref[...] = h + hu
    po_ref[...] = pos_ref[...] + num / jnp.maximum(cnt, 1.0)


def _node(h, pos, pm0, pm1, pp0, pp1, wn1a, wn1b, bn1, wn2, bn2):
    return pl.pallas_call(
        _node_body,
        grid=(N // BN,),
        in_specs=[
            pl.BlockSpec((BN, D), lambda i: (i, 0)),
            pl.BlockSpec((BN, 3), lambda i: (i, 0)),
            pl.BlockSpec((BN, D), lambda i: (i, 0)),
            pl.BlockSpec((BN, D), lambda i: (i, 0)),
            pl.BlockSpec((BN, 4), lambda i: (i, 0)),
            pl.BlockSpec((BN, 4), lambda i: (i, 0)),
            pl.BlockSpec((D, D), lambda i: (0, 0)),
            pl.BlockSpec((D, D), lambda i: (0, 0)),
            pl.BlockSpec((1, D), lambda i: (0, 0)),
            pl.BlockSpec((D, D), lambda i: (0, 0)),
            pl.BlockSpec((1, D), lambda i: (0, 0)),
        ],
        out_specs=[
            pl.BlockSpec((BN, D), lambda i: (i, 0)),
            pl.BlockSpec((BN, 3), lambda i: (i, 0)),
        ],
        out_shape=[jax.ShapeDtypeStruct((N, D), jnp.float32),
                   jax.ShapeDtypeStruct((N, 3), jnp.float32)],
    )(h, pos, pm0, pm1, pp0, pp1, wn1a, wn1b, bn1, wn2, bn2)


def kernel(h, pos, edge_index, W_e1, b_e1, W_e2, b_e2, W_c1, b_c1, W_c2,
           W_n1, b_n1, W_n2, b_n2):
    row = edge_index[0].astype(jnp.int32)
    col = edge_index[1].astype(jnp.int32)
    px = pos[:, 0]
    py = pos[:, 1]
    pz = pos[:, 2]

    t1, t2 = _prep(h, W_e1[:D], W_e1[D:2 * D], b_e1.reshape(1, D))
    g, dxa, dya, dza, sqa = _sc_gather(t1, t2, row, col, px, py, pz)
    m, pux, puy, puz = _edge(
        g, dxa.reshape(EB, 1, BE), dya.reshape(EB, 1, BE),
        dza.reshape(EB, 1, BE), sqa.reshape(EB, 1, BE),
        W_e1[2 * D:2 * D + 1], W_e2, b_e2.reshape(1, D),
        W_c1, b_c1.reshape(1, D), W_c2)
    pm = _sc_scatter(m, row, jnp.zeros((NPAD, D), jnp.float32))
    pp = _sc_pos_scatter(row, pux.reshape(E), puy.reshape(E),
                         puz.reshape(E), jnp.zeros((P4,), jnp.float32))
    pp = pp.reshape(NC, NPAD, 4)
    h_out, pos_out = _node(h, pos, pm[0], pm[1], pp[0], pp[1],
                           W_n1[:D], W_n1[D:], b_n1.reshape(1, D),
                           W_n2, b_n2.reshape(1, D))
    return h_out, pos_out
